# baseline (speedup 1.0000x reference)
import functools

import jax
import jax.numpy as jnp
from jax import lax
from jax.experimental import pallas as pl
from jax.experimental.pallas import tpu as pltpu
from jax.experimental.pallas import tpu_sc as plsc

N = 10000
E = 320000
D = 128
R = 8
RELD = 32
B = 100
L = 2

NC = 2
NS = 16
NW = NC * NS

NP = 10240
K = 128
C = 80
EPW = C * K
EP = NW * EPW

_MESH = plsc.VectorSubcoreMesh(core_axis_name="c", subcore_axis_name="s")


@functools.partial(
    pl.kernel,
    out_type=jax.ShapeDtypeStruct((NW, NP), jnp.float32),
    mesh=_MESH,
    scratch_types=[
        pltpu.VMEM((EPW,), jnp.int32),
        pltpu.VMEM((NP,), jnp.float32),
    ],
    compiler_params=pltpu.CompilerParams(needs_layout_passes=False),
)
def _deg(dst_hbm, out_hbm, dst_v, deg_v):
    cid = lax.axis_index("c")
    sid = lax.axis_index("s")
    wid = sid * NC + cid
    pltpu.sync_copy(dst_hbm.at[pl.ds(wid * EPW, EPW)], dst_v)

    zeros = jnp.zeros((16,), jnp.float32)
    def zbody(i, _):
        deg_v[pl.ds(pl.multiple_of(i * 16, 16), 16)] = zeros
        return 0
    lax.fori_loop(0, NP // 16, zbody, 0)

    ones = jnp.ones((16,), jnp.float32)
    def body(i, _):
        idx = dst_v[pl.ds(pl.multiple_of(i * 16, 16), 16)]
        plsc.addupdate_scatter(deg_v, [idx], ones)
        return 0
    lax.fori_loop(0, EPW // 16, body, 0)

    pltpu.sync_copy(deg_v, out_hbm.at[wid])


@functools.partial(
    pl.kernel,
    out_type=jax.ShapeDtypeStruct((NC * NP, D), jnp.float32),
    mesh=_MESH,
    scratch_types=[
        pltpu.VMEM((C, K), jnp.int32),
        pltpu.VMEM((K,), jnp.int32),
        pltpu.VMEM((K,), jnp.int32),
        pltpu.VMEM((K, D), jnp.float32),
        pltpu.VMEM((K, D), jnp.float32),
        pltpu.VMEM_SHARED((NP, D), jnp.float32),
        pltpu.SemaphoreType.DMA,
        pltpu.SemaphoreType.DMA,
        pltpu.SemaphoreType.DMA,
        pltpu.SemaphoreType.DMA,
    ],
)
def _edge_pass(table_hbm, ci_hbm, dst_hbm, zeros_hbm, out_hbm,
               ci_v, dst0_v, dst1_v, rows0_v, rows1_v, acc_sh,
               semg0, semg1, semd0, semd1):
    cid = lax.axis_index("c")
    sid = lax.axis_index("s")
    wid = sid * NC + cid
    rpt = NP // NS

    pltpu.sync_copy(zeros_hbm.at[pl.ds(sid * rpt, rpt)],
                    acc_sh.at[pl.ds(sid * rpt, rpt)])
    pltpu.sync_copy(ci_hbm.at[wid], ci_v)
    plsc.subcore_barrier()

    rows = (rows0_v, rows1_v)
    dsts = (dst0_v, dst1_v)
    semg = (semg0, semg1)
    semd = (semd0, semd1)
    base = wid * EPW

    def start(j, b):
        pltpu.async_copy(dst_hbm.at[pl.ds(base + j * K, K)], dsts[b], semd[b])
        pltpu.async_copy(table_hbm.at[ci_v.at[j]], rows[b], semg[b])

    def finish(j, b):
        pltpu.make_async_copy(dst_hbm.at[pl.ds(base + j * K, K)], dsts[b],
                              semd[b]).wait()
        pltpu.make_async_copy(table_hbm.at[ci_v.at[j]], rows[b],
                              semg[b]).wait()
        pltpu.sync_copy(rows[b], acc_sh.at[dsts[b]], add=True)

    start(0, 0)

    def body(i, _):
        jj = i * 2
        for b in range(2):
            j = jj + b

            @pl.when(j + 1 < C)
            def _():
                start(j + 1, 1 - b)

            finish(j, b)
        return 0
    lax.fori_loop(0, C // 2, body, 0)

    plsc.subcore_barrier()
    pltpu.sync_copy(acc_sh.at[pl.ds(sid * rpt, rpt)],
                    out_hbm.at[pl.ds(cid * NP + sid * rpt, rpt)])


BN = 1280
NB = NP // BN


def _transform_body(h_ref, w_ref, out_ref):
    out_ref[0] = jnp.dot(h_ref[...].astype(jnp.bfloat16),
                         w_ref[0].astype(jnp.bfloat16),
                         preferred_element_type=jnp.float32)


_transform = pl.pallas_call(
    _transform_body,
    grid=(NB, R),
    in_specs=[
        pl.BlockSpec((BN, D), lambda i, r: (i, 0)),
        pl.BlockSpec((1, D, D), lambda i, r: (r, 0, 0)),
    ],
    out_specs=pl.BlockSpec((1, BN, D), lambda i, r: (r, i, 0)),
    out_shape=jax.ShapeDtypeStruct((R, NP, D), jnp.float32),
)


def _combine_body(acc_ref, h_ref, wself_ref, deg_ref, out_ref):
    deg = jnp.sum(deg_ref[...], axis=0)
    norm = 1.0 / jnp.where(deg == 0.0, 1.0, deg)
    a = acc_ref[0] + acc_ref[1]
    hw = jnp.dot(h_ref[...], wself_ref[...],
                 preferred_element_type=jnp.float32)
    out_ref[...] = jnp.maximum(a * norm[:, None] + hw, 0.0)


_combine = pl.pallas_call(
    _combine_body,
    grid=(NB,),
    in_specs=[
        pl.BlockSpec((NC, BN, D), lambda i: (0, i, 0)),
        pl.BlockSpec((BN, D), lambda i: (i, 0)),
        pl.BlockSpec((D, D), lambda i: (0, 0)),
        pl.BlockSpec((NW, BN), lambda i: (0, i)),
    ],
    out_specs=pl.BlockSpec((BN, D), lambda i: (i, 0)),
    out_shape=jax.ShapeDtypeStruct((NP, D), jnp.float32),
)


def _tail_body(acc_ref, h_ref, wself_ref, deg_ref, seg_ref, head_ref,
               tail_ref, rlab_ref, rel_emb_ref, wr_ref, wcat_ref, fcb_ref,
               out_ref, sg, sh, st, sc):
    i = pl.program_id(0)
    deg = jnp.sum(deg_ref[...], axis=0)
    norm = 1.0 / jnp.where(deg == 0.0, 1.0, deg)
    a = acc_ref[0] + acc_ref[1]
    hw = jnp.dot(h_ref[...], wself_ref[...],
                 preferred_element_type=jnp.float32)
    repr_blk = jnp.maximum(a * norm[:, None] + hw, 0.0)
    u = jnp.dot(repr_blk, wcat_ref[...],
                preferred_element_type=jnp.float32)
    n_iota = lax.broadcasted_iota(jnp.int32, (B, BN), 1) + i * BN
    b_iota = lax.broadcasted_iota(jnp.int32, (B, BN), 0)
    segm = (seg_ref[...] == b_iota).astype(jnp.float32)
    headm = (head_ref[...] == n_iota).astype(jnp.float32)
    tailm = (tail_ref[...] == n_iota).astype(jnp.float32)
    pg = jnp.dot(segm, u, preferred_element_type=jnp.float32)
    ph = jnp.dot(headm, u, preferred_element_type=jnp.float32)
    pt = jnp.dot(tailm, u, preferred_element_type=jnp.float32)
    cnt = jnp.sum(segm, axis=1, keepdims=True)

    @pl.when(i == 0)
    def _():
        sg[...] = pg
        sh[...] = ph
        st[...] = pt
        sc[...] = cnt

    @pl.when(i > 0)
    def _():
        sg[...] += pg
        sh[...] += ph
        st[...] += pt
        sc[...] += cnt

    @pl.when(i == NB - 1)
    def _():
        r_iota = lax.broadcasted_iota(jnp.int32, (B, R), 1)
        relm = (rlab_ref[...] == r_iota).astype(jnp.float32)
        z = jnp.sum(rel_emb_ref[...] * wr_ref[...], axis=1,
                    keepdims=True)
        rp = jnp.dot(relm, z, preferred_element_type=jnp.float32)
        counts = jnp.maximum(sc[...], 1.0)
        out_ref[...] = sg[:, 0:1] / counts + sh[:, 1:2] + st[:, 2:3] + rp \
            + fcb_ref[...]


_tail = pl.pallas_call(
    _tail_body,
    grid=(NB,),
    in_specs=[
        pl.BlockSpec((NC, BN, D), lambda i: (0, i, 0)),
        pl.BlockSpec((BN, D), lambda i: (i, 0)),
        pl.BlockSpec((D, D), lambda i: (0, 0)),
        pl.BlockSpec((NW, BN), lambda i: (0, i)),
        pl.BlockSpec((1, BN), lambda i: (0, i)),
        pl.BlockSpec((B, 1), lambda i: (0, 0)),
        pl.BlockSpec((B, 1), lambda i: (0, 0)),
        pl.BlockSpec((B, 1), lambda i: (0, 0)),
        pl.BlockSpec((R, RELD), lambda i: (0, 0)),
        pl.BlockSpec((1, RELD), lambda i: (0, 0)),
        pl.BlockSpec((D, 128), lambda i: (0, 0)),
        pl.BlockSpec((1, 1), lambda i: (0, 0)),
    ],
    out_specs=pl.BlockSpec((B, 1), lambda i: (0, 0)),
    out_shape=jax.ShapeDtypeStruct((B, 1), jnp.float32),
    scratch_shapes=[
        pltpu.VMEM((B, 128), jnp.float32),
        pltpu.VMEM((B, 128), jnp.float32),
        pltpu.VMEM((B, 128), jnp.float32),
        pltpu.VMEM((B, 1), jnp.float32),
    ],
)


def kernel(x, edge_index, edge_type, segment_ids, head_ids, tail_ids,
           rel_labels, W_rel, W_self, rel_emb, fc_w, fc_b):
    src = edge_index[0].astype(jnp.int32)
    dst = edge_index[1].astype(jnp.int32)
    et = edge_type.astype(jnp.int32)

    pad_i = jnp.arange(EP - E, dtype=jnp.int32)
    ci = et * NP + src
    ci = jnp.concatenate([ci, pad_i % NP])
    dst_p = jnp.concatenate([dst, N + pad_i % (NP - N)])

    x_p = jnp.pad(x, ((0, NP - N), (0, 0)))
    seg_p = jnp.concatenate(
        [segment_ids.astype(jnp.int32), jnp.full((NP - N,), B, jnp.int32)]
    ).reshape(1, NP)
    deg = _deg(dst_p)
    zeros_init = jnp.broadcast_to(deg[:1, :1] * 0.0, (NP, D))

    h = x_p
    for l in range(L - 1):
        hw = _transform(h, W_rel[l])
        table = hw.reshape(R * NP, D)
        acc = _edge_pass(table, ci.reshape(NW, C, K), dst_p, zeros_init)
        h = _combine(acc.reshape(NC, NP, D), h, W_self[l], deg)

    hw = _transform(h, W_rel[L - 1])
    table = hw.reshape(R * NP, D)
    acc = _edge_pass(table, ci.reshape(NW, C, K), dst_p, zeros_init)

    wcat = jnp.pad(jnp.stack([fc_w[:D, 0], fc_w[D:2 * D, 0],
                              fc_w[2 * D:3 * D, 0]], axis=1),
                   ((0, 0), (0, D - 3)))
    wr = fc_w[3 * D:, 0].reshape(1, RELD)
    out = _tail(acc.reshape(NC, NP, D), h, W_self[L - 1], deg, seg_p,
                head_ids.astype(jnp.int32).reshape(B, 1),
                tail_ids.astype(jnp.int32).reshape(B, 1),
                rel_labels.astype(jnp.int32).reshape(B, 1),
                rel_emb, wr, wcat, fc_b.reshape(1, 1))
    return out


# speedup vs baseline: 1.1031x; 1.1031x over previous
import functools

import jax
import jax.numpy as jnp
from jax import lax
from jax.experimental import pallas as pl
from jax.experimental.pallas import tpu as pltpu
from jax.experimental.pallas import tpu_sc as plsc

N = 10000
E = 320000
D = 128
R = 8
RELD = 32
B = 100
L = 2

NC = 2
NS = 16
NW = NC * NS

NP = 10240
K = 128
C = 80
EPW = C * K
EP = NW * EPW

_MESH = plsc.VectorSubcoreMesh(core_axis_name="c", subcore_axis_name="s")


@functools.partial(
    pl.kernel,
    out_type=jax.ShapeDtypeStruct((NW, NP), jnp.float32),
    mesh=_MESH,
    scratch_types=[
        pltpu.VMEM((EPW,), jnp.int32),
        pltpu.VMEM((NP,), jnp.float32),
    ],
    compiler_params=pltpu.CompilerParams(needs_layout_passes=False),
)
def _deg(dst_hbm, out_hbm, dst_v, deg_v):
    cid = lax.axis_index("c")
    sid = lax.axis_index("s")
    wid = sid * NC + cid
    pltpu.sync_copy(dst_hbm.at[pl.ds(wid * EPW, EPW)], dst_v)

    zeros = jnp.zeros((16,), jnp.float32)
    def zbody(i, _):
        deg_v[pl.ds(pl.multiple_of(i * 16, 16), 16)] = zeros
        return 0
    lax.fori_loop(0, NP // 16, zbody, 0)

    ones = jnp.ones((16,), jnp.float32)
    def body(i, _):
        idx = dst_v[pl.ds(pl.multiple_of(i * 16, 16), 16)]
        plsc.addupdate_scatter(deg_v, [idx], ones)
        return 0
    lax.fori_loop(0, EPW // 16, body, 0)

    pltpu.sync_copy(deg_v, out_hbm.at[wid])


@functools.partial(
    pl.kernel,
    out_type=jax.ShapeDtypeStruct((NC * NP, D), jnp.float32),
    mesh=_MESH,
    scratch_types=[
        pltpu.VMEM((C, K), jnp.int32),
        pltpu.VMEM((K,), jnp.int32),
        pltpu.VMEM((K,), jnp.int32),
        pltpu.VMEM((K, D), jnp.float32),
        pltpu.VMEM((K, D), jnp.float32),
        pltpu.VMEM_SHARED((NP, D), jnp.float32),
        pltpu.SemaphoreType.DMA,
        pltpu.SemaphoreType.DMA,
        pltpu.SemaphoreType.DMA,
        pltpu.SemaphoreType.DMA,
    ],
)
def _edge_pass(table_hbm, ci_hbm, dst_hbm, zeros_hbm, out_hbm,
               ci_v, dst0_v, dst1_v, rows0_v, rows1_v, acc_sh,
               semg0, semg1, semd0, semd1):
    cid = lax.axis_index("c")
    sid = lax.axis_index("s")
    wid = sid * NC + cid
    rpt = NP // NS

    pltpu.sync_copy(zeros_hbm.at[pl.ds(sid * rpt, rpt)],
                    acc_sh.at[pl.ds(sid * rpt, rpt)])
    pltpu.sync_copy(ci_hbm.at[wid], ci_v)
    plsc.subcore_barrier()

    rows = (rows0_v, rows1_v)
    dsts = (dst0_v, dst1_v)
    semg = (semg0, semg1)
    semd = (semd0, semd1)
    base = wid * EPW

    def start(j, b):
        pltpu.async_copy(dst_hbm.at[pl.ds(base + j * K, K)], dsts[b], semd[b])
        pltpu.async_copy(table_hbm.at[ci_v.at[j]], rows[b], semg[b])

    def finish(j, b):
        pltpu.make_async_copy(dst_hbm.at[pl.ds(base + j * K, K)], dsts[b],
                              semd[b]).wait()
        pltpu.make_async_copy(table_hbm.at[ci_v.at[j]], rows[b],
                              semg[b]).wait()
        pltpu.sync_copy(rows[b], acc_sh.at[dsts[b]], add=True)

    start(0, 0)

    def body(i, _):
        jj = i * 2
        for b in range(2):
            j = jj + b

            @pl.when(j + 1 < C)
            def _():
                start(j + 1, 1 - b)

            finish(j, b)
        return 0
    lax.fori_loop(0, C // 2, body, 0)

    plsc.subcore_barrier()
    pltpu.sync_copy(acc_sh.at[pl.ds(sid * rpt, rpt)],
                    out_hbm.at[pl.ds(cid * NP + sid * rpt, rpt)])


BN = 1280
NB = NP // BN


def _transform_body(h_ref, w_ref, out_ref):
    out_ref[0] = jnp.dot(h_ref[...], w_ref[0],
                         preferred_element_type=jnp.float32)


TN = 2560
_transform = pl.pallas_call(
    _transform_body,
    grid=(NP // TN, R),
    in_specs=[
        pl.BlockSpec((TN, D), lambda i, r: (i, 0)),
        pl.BlockSpec((1, D, D), lambda i, r: (r, 0, 0)),
    ],
    out_specs=pl.BlockSpec((1, TN, D), lambda i, r: (r, i, 0)),
    out_shape=jax.ShapeDtypeStruct((R, NP, D), jnp.float32),
)


def _combine_body(acc_ref, h_ref, wself_ref, deg_ref, out_ref):
    deg = jnp.sum(deg_ref[...], axis=0)
    norm = 1.0 / jnp.where(deg == 0.0, 1.0, deg)
    a = acc_ref[0] + acc_ref[1]
    hw = jnp.dot(h_ref[...], wself_ref[...],
                 preferred_element_type=jnp.float32)
    out_ref[...] = jnp.maximum(a * norm[:, None] + hw, 0.0)


_combine = pl.pallas_call(
    _combine_body,
    grid=(NB,),
    in_specs=[
        pl.BlockSpec((NC, BN, D), lambda i: (0, i, 0)),
        pl.BlockSpec((BN, D), lambda i: (i, 0)),
        pl.BlockSpec((D, D), lambda i: (0, 0)),
        pl.BlockSpec((NW, BN), lambda i: (0, i)),
    ],
    out_specs=pl.BlockSpec((BN, D), lambda i: (i, 0)),
    out_shape=jax.ShapeDtypeStruct((NP, D), jnp.float32),
)


def _tail_body(acc_ref, h_ref, wself_ref, deg_ref, seg_ref, head_ref,
               tail_ref, rlab_ref, rel_emb_ref, wr_ref, wcat_ref, fcb_ref,
               out_ref, sg, sh, st, sc):
    i = pl.program_id(0)
    deg = jnp.sum(deg_ref[...], axis=0)
    norm = 1.0 / jnp.where(deg == 0.0, 1.0, deg)
    a = acc_ref[0] + acc_ref[1]
    hw = jnp.dot(h_ref[...], wself_ref[...],
                 preferred_element_type=jnp.float32)
    repr_blk = jnp.maximum(a * norm[:, None] + hw, 0.0)
    u = jnp.dot(repr_blk, wcat_ref[...],
                preferred_element_type=jnp.float32)
    n_iota = lax.broadcasted_iota(jnp.int32, (B, BN), 1) + i * BN
    b_iota = lax.broadcasted_iota(jnp.int32, (B, BN), 0)
    segm = (seg_ref[...] == b_iota).astype(jnp.float32)
    headm = (head_ref[...] == n_iota).astype(jnp.float32)
    tailm = (tail_ref[...] == n_iota).astype(jnp.float32)
    pg = jnp.dot(segm, u, preferred_element_type=jnp.float32)
    ph = jnp.dot(headm, u, preferred_element_type=jnp.float32)
    pt = jnp.dot(tailm, u, preferred_element_type=jnp.float32)
    cnt = jnp.sum(segm, axis=1, keepdims=True)

    @pl.when(i == 0)
    def _():
        sg[...] = pg
        sh[...] = ph
        st[...] = pt
        sc[...] = cnt

    @pl.when(i > 0)
    def _():
        sg[...] += pg
        sh[...] += ph
        st[...] += pt
        sc[...] += cnt

    @pl.when(i == NB - 1)
    def _():
        r_iota = lax.broadcasted_iota(jnp.int32, (B, R), 1)
        relm = (rlab_ref[...] == r_iota).astype(jnp.float32)
        z = jnp.sum(rel_emb_ref[...] * wr_ref[...], axis=1,
                    keepdims=True)
        rp = jnp.dot(relm, z, preferred_element_type=jnp.float32)
        counts = jnp.maximum(sc[...], 1.0)
        out_ref[...] = sg[:, 0:1] / counts + sh[:, 1:2] + st[:, 2:3] + rp \
            + fcb_ref[...]


_tail = pl.pallas_call(
    _tail_body,
    grid=(NB,),
    in_specs=[
        pl.BlockSpec((NC, BN, D), lambda i: (0, i, 0)),
        pl.BlockSpec((BN, D), lambda i: (i, 0)),
        pl.BlockSpec((D, D), lambda i: (0, 0)),
        pl.BlockSpec((NW, BN), lambda i: (0, i)),
        pl.BlockSpec((1, BN), lambda i: (0, i)),
        pl.BlockSpec((B, 1), lambda i: (0, 0)),
        pl.BlockSpec((B, 1), lambda i: (0, 0)),
        pl.BlockSpec((B, 1), lambda i: (0, 0)),
        pl.BlockSpec((R, RELD), lambda i: (0, 0)),
        pl.BlockSpec((1, RELD), lambda i: (0, 0)),
        pl.BlockSpec((D, 128), lambda i: (0, 0)),
        pl.BlockSpec((1, 1), lambda i: (0, 0)),
    ],
    out_specs=pl.BlockSpec((B, 1), lambda i: (0, 0)),
    out_shape=jax.ShapeDtypeStruct((B, 1), jnp.float32),
    scratch_shapes=[
        pltpu.VMEM((B, 128), jnp.float32),
        pltpu.VMEM((B, 128), jnp.float32),
        pltpu.VMEM((B, 128), jnp.float32),
        pltpu.VMEM((B, 1), jnp.float32),
    ],
)


def kernel(x, edge_index, edge_type, segment_ids, head_ids, tail_ids,
           rel_labels, W_rel, W_self, rel_emb, fc_w, fc_b):
    src = edge_index[0].astype(jnp.int32)
    dst = edge_index[1].astype(jnp.int32)
    et = edge_type.astype(jnp.int32)

    pad_i = jnp.arange(EP - E, dtype=jnp.int32)
    ci = et * NP + src
    ci = jnp.concatenate([ci, pad_i % NP])
    dst_p = jnp.concatenate([dst, N + pad_i % (NP - N)])

    x_p = jnp.pad(x, ((0, NP - N), (0, 0)))
    seg_p = jnp.concatenate(
        [segment_ids.astype(jnp.int32), jnp.full((NP - N,), B, jnp.int32)]
    ).reshape(1, NP)
    deg = _deg(dst_p)
    zeros_init = jnp.broadcast_to(deg[:1, :1] * 0.0, (NP, D))

    h = x_p
    for l in range(L - 1):
        hw = _transform(h, W_rel[l])
        table = hw.reshape(R * NP, D)
        acc = _edge_pass(table, ci.reshape(NW, C, K), dst_p, zeros_init)
        h = _combine(acc.reshape(NC, NP, D), h, W_self[l], deg)

    hw = _transform(h, W_rel[L - 1])
    table = hw.reshape(R * NP, D)
    acc = _edge_pass(table, ci.reshape(NW, C, K), dst_p, zeros_init)

    wcat = jnp.pad(jnp.stack([fc_w[:D, 0], fc_w[D:2 * D, 0],
                              fc_w[2 * D:3 * D, 0]], axis=1),
                   ((0, 0), (0, D - 3)))
    wr = fc_w[3 * D:, 0].reshape(1, RELD)
    out = _tail(acc.reshape(NC, NP, D), h, W_self[L - 1], deg, seg_p,
                head_ids.astype(jnp.int32).reshape(B, 1),
                tail_ids.astype(jnp.int32).reshape(B, 1),
                rel_labels.astype(jnp.int32).reshape(B, 1),
                rel_emb, wr, wcat, fc_b.reshape(1, 1))
    return out


# speedup vs baseline: 1.1637x; 1.0550x over previous
import functools

import jax
import jax.numpy as jnp
from jax import lax
from jax.experimental import pallas as pl
from jax.experimental.pallas import tpu as pltpu
from jax.experimental.pallas import tpu_sc as plsc

N = 10000
E = 320000
D = 128
R = 8
RELD = 32
B = 100
L = 2

NC = 2
NS = 16
NW = NC * NS

NP = 10240
K = 128
C = 80
EPW = C * K
EP = NW * EPW

_MESH = plsc.VectorSubcoreMesh(core_axis_name="c", subcore_axis_name="s")


@functools.partial(
    pl.kernel,
    out_type=jax.ShapeDtypeStruct((NW, NP), jnp.float32),
    mesh=_MESH,
    scratch_types=[
        pltpu.VMEM((EPW,), jnp.int32),
        pltpu.VMEM((NP,), jnp.float32),
    ],
    compiler_params=pltpu.CompilerParams(needs_layout_passes=False),
)
def _deg(dst_hbm, out_hbm, dst_v, deg_v):
    cid = lax.axis_index("c")
    sid = lax.axis_index("s")
    wid = sid * NC + cid
    pltpu.sync_copy(dst_hbm.at[pl.ds(wid * EPW, EPW)], dst_v)

    zeros = jnp.zeros((16,), jnp.float32)
    def zbody(i, _):
        deg_v[pl.ds(pl.multiple_of(i * 16, 16), 16)] = zeros
        return 0
    lax.fori_loop(0, NP // 16, zbody, 0)

    ones = jnp.ones((16,), jnp.float32)
    def body(i, _):
        idx = dst_v[pl.ds(pl.multiple_of(i * 16, 16), 16)]
        plsc.addupdate_scatter(deg_v, [idx], ones)
        return 0
    lax.fori_loop(0, EPW // 16, body, 0)

    pltpu.sync_copy(deg_v, out_hbm.at[wid])


@functools.partial(
    pl.kernel,
    out_type=jax.ShapeDtypeStruct((NC * NP, D), jnp.float32),
    mesh=_MESH,
    scratch_types=[
        pltpu.VMEM((C, K), jnp.int32),
        pltpu.VMEM((K,), jnp.int32),
        pltpu.VMEM((K,), jnp.int32),
        pltpu.VMEM((K, D), jnp.float32),
        pltpu.VMEM((K, D), jnp.float32),
        pltpu.VMEM_SHARED((NP, D), jnp.float32),
        pltpu.SemaphoreType.DMA,
        pltpu.SemaphoreType.DMA,
        pltpu.SemaphoreType.DMA,
        pltpu.SemaphoreType.DMA,
    ],
)
def _edge_pass(table_hbm, ci_hbm, dst_hbm, zeros_hbm, out_hbm,
               ci_v, dst0_v, dst1_v, rows0_v, rows1_v, acc_sh,
               semg0, semg1, semd0, semd1):
    cid = lax.axis_index("c")
    sid = lax.axis_index("s")
    wid = sid * NC + cid
    rpt = NP // NS

    pltpu.sync_copy(zeros_hbm.at[pl.ds(sid * rpt, rpt)],
                    acc_sh.at[pl.ds(sid * rpt, rpt)])
    pltpu.sync_copy(ci_hbm.at[wid], ci_v)
    plsc.subcore_barrier()

    rows = (rows0_v, rows1_v)
    dsts = (dst0_v, dst1_v)
    semg = (semg0, semg1)
    semd = (semd0, semd1)
    base = wid * EPW

    def start(j, b):
        pltpu.async_copy(dst_hbm.at[pl.ds(base + j * K, K)], dsts[b], semd[b])
        pltpu.async_copy(table_hbm.at[ci_v.at[j]], rows[b], semg[b])

    def finish(j, b):
        pltpu.make_async_copy(dst_hbm.at[pl.ds(base + j * K, K)], dsts[b],
                              semd[b]).wait()
        pltpu.make_async_copy(table_hbm.at[ci_v.at[j]], rows[b],
                              semg[b]).wait()
        pltpu.sync_copy(rows[b], acc_sh.at[dsts[b]], add=True)

    start(0, 0)

    def body(i, _):
        jj = i * 2
        for b in range(2):
            j = jj + b

            @pl.when(j + 1 < C)
            def _():
                start(j + 1, 1 - b)

            finish(j, b)
        return 0
    lax.fori_loop(0, C // 2, body, 0)

    plsc.subcore_barrier()
    pltpu.sync_copy(acc_sh.at[pl.ds(sid * rpt, rpt)],
                    out_hbm.at[pl.ds(cid * NP + sid * rpt, rpt)])


BN = 1280
NB = NP // BN


def _transform_body(h_ref, w_ref, out_ref):
    out_ref[0] = jnp.dot(h_ref[...], w_ref[0],
                         preferred_element_type=jnp.float32)


TN = 5120
_transform = pl.pallas_call(
    _transform_body,
    grid=(NP // TN, R),
    in_specs=[
        pl.BlockSpec((TN, D), lambda i, r: (i, 0)),
        pl.BlockSpec((1, D, D), lambda i, r: (r, 0, 0)),
    ],
    out_specs=pl.BlockSpec((1, TN, D), lambda i, r: (r, i, 0)),
    out_shape=jax.ShapeDtypeStruct((R, NP, D), jnp.float32),
)


def _combine_body(acc_ref, h_ref, wself_ref, deg_ref, out_ref):
    deg = jnp.sum(deg_ref[...], axis=0)
    norm = 1.0 / jnp.where(deg == 0.0, 1.0, deg)
    a = acc_ref[0] + acc_ref[1]
    hw = jnp.dot(h_ref[...], wself_ref[...],
                 preferred_element_type=jnp.float32)
    out_ref[...] = jnp.maximum(a * norm[:, None] + hw, 0.0)


_combine = pl.pallas_call(
    _combine_body,
    grid=(NB,),
    in_specs=[
        pl.BlockSpec((NC, BN, D), lambda i: (0, i, 0)),
        pl.BlockSpec((BN, D), lambda i: (i, 0)),
        pl.BlockSpec((D, D), lambda i: (0, 0)),
        pl.BlockSpec((NW, BN), lambda i: (0, i)),
    ],
    out_specs=pl.BlockSpec((BN, D), lambda i: (i, 0)),
    out_shape=jax.ShapeDtypeStruct((NP, D), jnp.float32),
)


def _tail_body(acc_ref, h_ref, wself_ref, deg_ref, seg_ref, head_ref,
               tail_ref, rlab_ref, rel_emb_ref, wr_ref, wcat_ref, fcb_ref,
               out_ref, sg, sh, st, sc):
    i = pl.program_id(0)
    deg = jnp.sum(deg_ref[...], axis=0)
    norm = 1.0 / jnp.where(deg == 0.0, 1.0, deg)
    a = acc_ref[0] + acc_ref[1]
    hw = jnp.dot(h_ref[...], wself_ref[...],
                 preferred_element_type=jnp.float32)
    repr_blk = jnp.maximum(a * norm[:, None] + hw, 0.0)
    u = jnp.dot(repr_blk, wcat_ref[...],
                preferred_element_type=jnp.float32)
    n_iota = lax.broadcasted_iota(jnp.int32, (B, BN), 1) + i * BN
    b_iota = lax.broadcasted_iota(jnp.int32, (B, BN), 0)
    segm = (seg_ref[...] == b_iota).astype(jnp.float32)
    headm = (head_ref[...] == n_iota).astype(jnp.float32)
    tailm = (tail_ref[...] == n_iota).astype(jnp.float32)
    pg = jnp.dot(segm, u, preferred_element_type=jnp.float32)
    ph = jnp.dot(headm, u, preferred_element_type=jnp.float32)
    pt = jnp.dot(tailm, u, preferred_element_type=jnp.float32)
    cnt = jnp.sum(segm, axis=1, keepdims=True)

    @pl.when(i == 0)
    def _():
        sg[...] = pg
        sh[...] = ph
        st[...] = pt
        sc[...] = cnt

    @pl.when(i > 0)
    def _():
        sg[...] += pg
        sh[...] += ph
        st[...] += pt
        sc[...] += cnt

    @pl.when(i == NB - 1)
    def _():
        r_iota = lax.broadcasted_iota(jnp.int32, (B, R), 1)
        relm = (rlab_ref[...] == r_iota).astype(jnp.float32)
        z = jnp.sum(rel_emb_ref[...] * wr_ref[...], axis=1,
                    keepdims=True)
        rp = jnp.dot(relm, z, preferred_element_type=jnp.float32)
        counts = jnp.maximum(sc[...], 1.0)
        out_ref[...] = sg[:, 0:1] / counts + sh[:, 1:2] + st[:, 2:3] + rp \
            + fcb_ref[...]


_tail = pl.pallas_call(
    _tail_body,
    grid=(NB,),
    in_specs=[
        pl.BlockSpec((NC, BN, D), lambda i: (0, i, 0)),
        pl.BlockSpec((BN, D), lambda i: (i, 0)),
        pl.BlockSpec((D, D), lambda i: (0, 0)),
        pl.BlockSpec((NW, BN), lambda i: (0, i)),
        pl.BlockSpec((1, BN), lambda i: (0, i)),
        pl.BlockSpec((B, 1), lambda i: (0, 0)),
        pl.BlockSpec((B, 1), lambda i: (0, 0)),
        pl.BlockSpec((B, 1), lambda i: (0, 0)),
        pl.BlockSpec((R, RELD), lambda i: (0, 0)),
        pl.BlockSpec((1, RELD), lambda i: (0, 0)),
        pl.BlockSpec((D, 128), lambda i: (0, 0)),
        pl.BlockSpec((1, 1), lambda i: (0, 0)),
    ],
    out_specs=pl.BlockSpec((B, 1), lambda i: (0, 0)),
    out_shape=jax.ShapeDtypeStruct((B, 1), jnp.float32),
    scratch_shapes=[
        pltpu.VMEM((B, 128), jnp.float32),
        pltpu.VMEM((B, 128), jnp.float32),
        pltpu.VMEM((B, 128), jnp.float32),
        pltpu.VMEM((B, 1), jnp.float32),
    ],
)


def kernel(x, edge_index, edge_type, segment_ids, head_ids, tail_ids,
           rel_labels, W_rel, W_self, rel_emb, fc_w, fc_b):
    src = edge_index[0].astype(jnp.int32)
    dst = edge_index[1].astype(jnp.int32)
    et = edge_type.astype(jnp.int32)

    pad_i = jnp.arange(EP - E, dtype=jnp.int32)
    ci = et * NP + src
    ci = jnp.concatenate([ci, pad_i % NP])
    dst_p = jnp.concatenate([dst, N + pad_i % (NP - N)])

    x_p = jnp.pad(x, ((0, NP - N), (0, 0)))
    seg_p = jnp.concatenate(
        [segment_ids.astype(jnp.int32), jnp.full((NP - N,), B, jnp.int32)]
    ).reshape(1, NP)
    deg = _deg(dst_p)
    zeros_init = jnp.broadcast_to(deg[:1, :1] * 0.0, (NP, D))

    h = x_p
    for l in range(L - 1):
        hw = _transform(h, W_rel[l])
        table = hw.reshape(R * NP, D)
        acc = _edge_pass(table, ci.reshape(NW, C, K), dst_p, zeros_init)
        h = _combine(acc.reshape(NC, NP, D), h, W_self[l], deg)

    hw = _transform(h, W_rel[L - 1])
    table = hw.reshape(R * NP, D)
    acc = _edge_pass(table, ci.reshape(NW, C, K), dst_p, zeros_init)

    wcat = jnp.pad(jnp.stack([fc_w[:D, 0], fc_w[D:2 * D, 0],
                              fc_w[2 * D:3 * D, 0]], axis=1),
                   ((0, 0), (0, D - 3)))
    wr = fc_w[3 * D:, 0].reshape(1, RELD)
    out = _tail(acc.reshape(NC, NP, D), h, W_self[L - 1], deg, seg_p,
                head_ids.astype(jnp.int32).reshape(B, 1),
                tail_ids.astype(jnp.int32).reshape(B, 1),
                rel_labels.astype(jnp.int32).reshape(B, 1),
                rel_emb, wr, wcat, fc_b.reshape(1, 1))
    return out


# speedup vs baseline: 1.2005x; 1.0316x over previous
import functools

import jax
import jax.numpy as jnp
from jax import lax
from jax.experimental import pallas as pl
from jax.experimental.pallas import tpu as pltpu
from jax.experimental.pallas import tpu_sc as plsc

N = 10000
E = 320000
D = 128
R = 8
RELD = 32
B = 100
L = 2

NC = 2
NS = 16
NW = NC * NS

NP = 10240
K = 128
C = 80
EPW = C * K
EP = NW * EPW

_MESH = plsc.VectorSubcoreMesh(core_axis_name="c", subcore_axis_name="s")


@functools.partial(
    pl.kernel,
    out_type=jax.ShapeDtypeStruct((NW, NP), jnp.float32),
    mesh=_MESH,
    scratch_types=[
        pltpu.VMEM((EPW,), jnp.int32),
        pltpu.VMEM((NP,), jnp.float32),
    ],
    compiler_params=pltpu.CompilerParams(needs_layout_passes=False),
)
def _deg(dst_hbm, out_hbm, dst_v, deg_v):
    cid = lax.axis_index("c")
    sid = lax.axis_index("s")
    wid = sid * NC + cid
    pltpu.sync_copy(dst_hbm.at[pl.ds(wid * EPW, EPW)], dst_v)

    zeros = jnp.zeros((16,), jnp.float32)
    def zbody(i, _):
        deg_v[pl.ds(pl.multiple_of(i * 16, 16), 16)] = zeros
        return 0
    lax.fori_loop(0, NP // 16, zbody, 0)

    ones = jnp.ones((16,), jnp.float32)
    def body(i, _):
        idx = dst_v[pl.ds(pl.multiple_of(i * 16, 16), 16)]
        plsc.addupdate_scatter(deg_v, [idx], ones)
        return 0
    lax.fori_loop(0, EPW // 16, body, 0)

    pltpu.sync_copy(deg_v, out_hbm.at[wid])


@functools.partial(
    pl.kernel,
    out_type=jax.ShapeDtypeStruct((NC * NP, D), jnp.float32),
    mesh=_MESH,
    scratch_types=[
        pltpu.VMEM((C, K), jnp.int32),
        pltpu.VMEM((K,), jnp.int32),
        pltpu.VMEM((K,), jnp.int32),
        pltpu.VMEM((K, D), jnp.float32),
        pltpu.VMEM((K, D), jnp.float32),
        pltpu.VMEM_SHARED((NP, D), jnp.float32),
        pltpu.SemaphoreType.DMA,
        pltpu.SemaphoreType.DMA,
        pltpu.SemaphoreType.DMA,
        pltpu.SemaphoreType.DMA,
    ],
)
def _edge_pass(table_hbm, ci_hbm, dst_hbm, zeros_hbm, out_hbm,
               ci_v, dst0_v, dst1_v, rows0_v, rows1_v, acc_sh,
               semg0, semg1, semd0, semd1):
    cid = lax.axis_index("c")
    sid = lax.axis_index("s")
    wid = sid * NC + cid
    rpt = NP // NS

    pltpu.sync_copy(zeros_hbm.at[pl.ds(sid * rpt, rpt)],
                    acc_sh.at[pl.ds(sid * rpt, rpt)])
    pltpu.sync_copy(ci_hbm.at[wid], ci_v)
    plsc.subcore_barrier()

    rows = (rows0_v, rows1_v)
    dsts = (dst0_v, dst1_v)
    semg = (semg0, semg1)
    semd = (semd0, semd1)
    base = wid * EPW

    def start(j, b):
        pltpu.async_copy(dst_hbm.at[pl.ds(base + j * K, K)], dsts[b], semd[b])
        pltpu.async_copy(table_hbm.at[ci_v.at[j]], rows[b], semg[b])

    def finish(j, b):
        pltpu.make_async_copy(dst_hbm.at[pl.ds(base + j * K, K)], dsts[b],
                              semd[b]).wait()
        pltpu.make_async_copy(table_hbm.at[ci_v.at[j]], rows[b],
                              semg[b]).wait()
        pltpu.sync_copy(rows[b], acc_sh.at[dsts[b]], add=True)

    start(0, 0)

    def body(i, _):
        jj = i * 2
        for b in range(2):
            j = jj + b

            @pl.when(j + 1 < C)
            def _():
                start(j + 1, 1 - b)

            finish(j, b)
        return 0
    lax.fori_loop(0, C // 2, body, 0)

    plsc.subcore_barrier()
    pltpu.sync_copy(acc_sh.at[pl.ds(sid * rpt, rpt)],
                    out_hbm.at[pl.ds(cid * NP + sid * rpt, rpt)])


BN = 1280
NB = NP // BN


def _transform_body(h_ref, w_ref, out_ref):
    out_ref[0] = jnp.dot(h_ref[...], w_ref[0],
                         preferred_element_type=jnp.float32)


TN = 10240
_transform = pl.pallas_call(
    _transform_body,
    grid=(NP // TN, R),
    in_specs=[
        pl.BlockSpec((TN, D), lambda i, r: (i, 0)),
        pl.BlockSpec((1, D, D), lambda i, r: (r, 0, 0)),
    ],
    out_specs=pl.BlockSpec((1, TN, D), lambda i, r: (r, i, 0)),
    out_shape=jax.ShapeDtypeStruct((R, NP, D), jnp.float32),
)


def _combine_body(acc_ref, h_ref, wself_ref, deg_ref, out_ref):
    deg = jnp.sum(deg_ref[...], axis=0)
    norm = 1.0 / jnp.where(deg == 0.0, 1.0, deg)
    a = acc_ref[0] + acc_ref[1]
    hw = jnp.dot(h_ref[...], wself_ref[...],
                 preferred_element_type=jnp.float32)
    out_ref[...] = jnp.maximum(a * norm[:, None] + hw, 0.0)


_combine = pl.pallas_call(
    _combine_body,
    grid=(NB,),
    in_specs=[
        pl.BlockSpec((NC, BN, D), lambda i: (0, i, 0)),
        pl.BlockSpec((BN, D), lambda i: (i, 0)),
        pl.BlockSpec((D, D), lambda i: (0, 0)),
        pl.BlockSpec((NW, BN), lambda i: (0, i)),
    ],
    out_specs=pl.BlockSpec((BN, D), lambda i: (i, 0)),
    out_shape=jax.ShapeDtypeStruct((NP, D), jnp.float32),
)


def _tail_body(acc_ref, h_ref, wself_ref, deg_ref, seg_ref, head_ref,
               tail_ref, rlab_ref, rel_emb_ref, wr_ref, wcat_ref, fcb_ref,
               out_ref, sg, sh, st, sc):
    i = pl.program_id(0)
    deg = jnp.sum(deg_ref[...], axis=0)
    norm = 1.0 / jnp.where(deg == 0.0, 1.0, deg)
    a = acc_ref[0] + acc_ref[1]
    hw = jnp.dot(h_ref[...], wself_ref[...],
                 preferred_element_type=jnp.float32)
    repr_blk = jnp.maximum(a * norm[:, None] + hw, 0.0)
    u = jnp.dot(repr_blk, wcat_ref[...],
                preferred_element_type=jnp.float32)
    n_iota = lax.broadcasted_iota(jnp.int32, (B, BN), 1) + i * BN
    b_iota = lax.broadcasted_iota(jnp.int32, (B, BN), 0)
    segm = (seg_ref[...] == b_iota).astype(jnp.float32)
    headm = (head_ref[...] == n_iota).astype(jnp.float32)
    tailm = (tail_ref[...] == n_iota).astype(jnp.float32)
    pg = jnp.dot(segm, u, preferred_element_type=jnp.float32)
    ph = jnp.dot(headm, u, preferred_element_type=jnp.float32)
    pt = jnp.dot(tailm, u, preferred_element_type=jnp.float32)
    cnt = jnp.sum(segm, axis=1, keepdims=True)

    @pl.when(i == 0)
    def _():
        sg[...] = pg
        sh[...] = ph
        st[...] = pt
        sc[...] = cnt

    @pl.when(i > 0)
    def _():
        sg[...] += pg
        sh[...] += ph
        st[...] += pt
        sc[...] += cnt

    @pl.when(i == NB - 1)
    def _():
        r_iota = lax.broadcasted_iota(jnp.int32, (B, R), 1)
        relm = (rlab_ref[...] == r_iota).astype(jnp.float32)
        z = jnp.sum(rel_emb_ref[...] * wr_ref[...], axis=1,
                    keepdims=True)
        rp = jnp.dot(relm, z, preferred_element_type=jnp.float32)
        counts = jnp.maximum(sc[...], 1.0)
        out_ref[...] = sg[:, 0:1] / counts + sh[:, 1:2] + st[:, 2:3] + rp \
            + fcb_ref[...]


_tail = pl.pallas_call(
    _tail_body,
    grid=(NB,),
    in_specs=[
        pl.BlockSpec((NC, BN, D), lambda i: (0, i, 0)),
        pl.BlockSpec((BN, D), lambda i: (i, 0)),
        pl.BlockSpec((D, D), lambda i: (0, 0)),
        pl.BlockSpec((NW, BN), lambda i: (0, i)),
        pl.BlockSpec((1, BN), lambda i: (0, i)),
        pl.BlockSpec((B, 1), lambda i: (0, 0)),
        pl.BlockSpec((B, 1), lambda i: (0, 0)),
        pl.BlockSpec((B, 1), lambda i: (0, 0)),
        pl.BlockSpec((R, RELD), lambda i: (0, 0)),
        pl.BlockSpec((1, RELD), lambda i: (0, 0)),
        pl.BlockSpec((D, 128), lambda i: (0, 0)),
        pl.BlockSpec((1, 1), lambda i: (0, 0)),
    ],
    out_specs=pl.BlockSpec((B, 1), lambda i: (0, 0)),
    out_shape=jax.ShapeDtypeStruct((B, 1), jnp.float32),
    scratch_shapes=[
        pltpu.VMEM((B, 128), jnp.float32),
        pltpu.VMEM((B, 128), jnp.float32),
        pltpu.VMEM((B, 128), jnp.float32),
        pltpu.VMEM((B, 1), jnp.float32),
    ],
)


def kernel(x, edge_index, edge_type, segment_ids, head_ids, tail_ids,
           rel_labels, W_rel, W_self, rel_emb, fc_w, fc_b):
    src = edge_index[0].astype(jnp.int32)
    dst = edge_index[1].astype(jnp.int32)
    et = edge_type.astype(jnp.int32)

    pad_i = jnp.arange(EP - E, dtype=jnp.int32)
    ci = et * NP + src
    ci = jnp.concatenate([ci, pad_i % NP])
    dst_p = jnp.concatenate([dst, N + pad_i % (NP - N)])

    x_p = jnp.pad(x, ((0, NP - N), (0, 0)))
    seg_p = jnp.concatenate(
        [segment_ids.astype(jnp.int32), jnp.full((NP - N,), B, jnp.int32)]
    ).reshape(1, NP)
    deg = _deg(dst_p)
    zeros_init = jnp.broadcast_to(deg[:1, :1] * 0.0, (NP, D))

    h = x_p
    for l in range(L - 1):
        hw = _transform(h, W_rel[l])
        table = hw.reshape(R * NP, D)
        acc = _edge_pass(table, ci.reshape(NW, C, K), dst_p, zeros_init)
        h = _combine(acc.reshape(NC, NP, D), h, W_self[l], deg)

    hw = _transform(h, W_rel[L - 1])
    table = hw.reshape(R * NP, D)
    acc = _edge_pass(table, ci.reshape(NW, C, K), dst_p, zeros_init)

    wcat = jnp.pad(jnp.stack([fc_w[:D, 0], fc_w[D:2 * D, 0],
                              fc_w[2 * D:3 * D, 0]], axis=1),
                   ((0, 0), (0, D - 3)))
    wr = fc_w[3 * D:, 0].reshape(1, RELD)
    out = _tail(acc.reshape(NC, NP, D), h, W_self[L - 1], deg, seg_p,
                head_ids.astype(jnp.int32).reshape(B, 1),
                tail_ids.astype(jnp.int32).reshape(B, 1),
                rel_labels.astype(jnp.int32).reshape(B, 1),
                rel_emb, wr, wcat, fc_b.reshape(1, 1))
    return out


# speedup vs baseline: 1.2152x; 1.0122x over previous
import functools

import jax
import jax.numpy as jnp
from jax import lax
from jax.experimental import pallas as pl
from jax.experimental.pallas import tpu as pltpu
from jax.experimental.pallas import tpu_sc as plsc

N = 10000
E = 320000
D = 128
R = 8
RELD = 32
B = 100
L = 2

NC = 2
NS = 16
NW = NC * NS

NP = 10240
K = 128
C = 80
EPW = C * K
EP = NW * EPW

_MESH = plsc.VectorSubcoreMesh(core_axis_name="c", subcore_axis_name="s")


@functools.partial(
    pl.kernel,
    out_type=jax.ShapeDtypeStruct((NW, NP), jnp.float32),
    mesh=_MESH,
    scratch_types=[
        pltpu.VMEM((EPW,), jnp.int32),
        pltpu.VMEM((NP,), jnp.float32),
    ],
    compiler_params=pltpu.CompilerParams(needs_layout_passes=False),
)
def _deg(dst_hbm, out_hbm, dst_v, deg_v):
    cid = lax.axis_index("c")
    sid = lax.axis_index("s")
    wid = sid * NC + cid
    pltpu.sync_copy(dst_hbm.at[pl.ds(wid * EPW, EPW)], dst_v)

    zeros = jnp.zeros((16,), jnp.float32)
    def zbody(i, _):
        deg_v[pl.ds(pl.multiple_of(i * 16, 16), 16)] = zeros
        return 0
    lax.fori_loop(0, NP // 16, zbody, 0)

    ones = jnp.ones((16,), jnp.float32)
    def body(i, _):
        idx = dst_v[pl.ds(pl.multiple_of(i * 16, 16), 16)]
        plsc.addupdate_scatter(deg_v, [idx], ones)
        return 0
    lax.fori_loop(0, EPW // 16, body, 0)

    pltpu.sync_copy(deg_v, out_hbm.at[wid])


@functools.partial(
    pl.kernel,
    out_type=jax.ShapeDtypeStruct((NC * NP, D), jnp.float32),
    mesh=_MESH,
    scratch_types=[
        pltpu.VMEM((C, K), jnp.int32),
        pltpu.VMEM((K,), jnp.int32),
        pltpu.VMEM((K,), jnp.int32),
        pltpu.VMEM((K, D), jnp.float32),
        pltpu.VMEM((K, D), jnp.float32),
        pltpu.VMEM_SHARED((NP, D), jnp.float32),
        pltpu.SemaphoreType.DMA,
        pltpu.SemaphoreType.DMA,
        pltpu.SemaphoreType.DMA,
        pltpu.SemaphoreType.DMA,
    ],
)
def _edge_pass(table_hbm, ci_hbm, dst_hbm, zeros_hbm, out_hbm,
               ci_v, dst0_v, dst1_v, rows0_v, rows1_v, acc_sh,
               semg0, semg1, semd0, semd1):
    cid = lax.axis_index("c")
    sid = lax.axis_index("s")
    wid = sid * NC + cid
    rpt = NP // NS

    pltpu.sync_copy(zeros_hbm.at[pl.ds(sid * rpt, rpt)],
                    acc_sh.at[pl.ds(sid * rpt, rpt)])
    pltpu.sync_copy(ci_hbm.at[wid], ci_v)
    plsc.subcore_barrier()

    rows = (rows0_v, rows1_v)
    dsts = (dst0_v, dst1_v)
    semg = (semg0, semg1)
    semd = (semd0, semd1)
    base = wid * EPW

    def start(j, b):
        pltpu.async_copy(dst_hbm.at[pl.ds(base + j * K, K)], dsts[b], semd[b])
        pltpu.async_copy(table_hbm.at[ci_v.at[j]], rows[b], semg[b])

    def finish(j, b):
        pltpu.make_async_copy(dst_hbm.at[pl.ds(base + j * K, K)], dsts[b],
                              semd[b]).wait()
        pltpu.make_async_copy(table_hbm.at[ci_v.at[j]], rows[b],
                              semg[b]).wait()
        pltpu.sync_copy(rows[b], acc_sh.at[dsts[b]], add=True)

    start(0, 0)

    def body(i, _):
        jj = i * 2
        for b in range(2):
            j = jj + b

            @pl.when(j + 1 < C)
            def _():
                start(j + 1, 1 - b)

            finish(j, b)
        return 0
    lax.fori_loop(0, C // 2, body, 0)

    plsc.subcore_barrier()
    pltpu.sync_copy(acc_sh.at[pl.ds(sid * rpt, rpt)],
                    out_hbm.at[pl.ds(cid * NP + sid * rpt, rpt)])


BN = 2560
NB = NP // BN


def _transform_body(h_ref, w_ref, out_ref):
    out_ref[0] = jnp.dot(h_ref[...], w_ref[0],
                         preferred_element_type=jnp.float32)


TN = 10240
_transform = pl.pallas_call(
    _transform_body,
    grid=(NP // TN, R),
    in_specs=[
        pl.BlockSpec((TN, D), lambda i, r: (i, 0)),
        pl.BlockSpec((1, D, D), lambda i, r: (r, 0, 0)),
    ],
    out_specs=pl.BlockSpec((1, TN, D), lambda i, r: (r, i, 0)),
    out_shape=jax.ShapeDtypeStruct((R, NP, D), jnp.float32),
)


def _combine_body(acc_ref, h_ref, wself_ref, deg_ref, out_ref):
    deg = jnp.sum(deg_ref[...], axis=0)
    norm = 1.0 / jnp.where(deg == 0.0, 1.0, deg)
    a = acc_ref[0] + acc_ref[1]
    hw = jnp.dot(h_ref[...], wself_ref[...],
                 preferred_element_type=jnp.float32)
    out_ref[...] = jnp.maximum(a * norm[:, None] + hw, 0.0)


_combine = pl.pallas_call(
    _combine_body,
    grid=(NB,),
    in_specs=[
        pl.BlockSpec((NC, BN, D), lambda i: (0, i, 0)),
        pl.BlockSpec((BN, D), lambda i: (i, 0)),
        pl.BlockSpec((D, D), lambda i: (0, 0)),
        pl.BlockSpec((NW, BN), lambda i: (0, i)),
    ],
    out_specs=pl.BlockSpec((BN, D), lambda i: (i, 0)),
    out_shape=jax.ShapeDtypeStruct((NP, D), jnp.float32),
)


def _tail_body(acc_ref, h_ref, wself_ref, deg_ref, seg_ref, head_ref,
               tail_ref, rlab_ref, rel_emb_ref, wr_ref, wcat_ref, fcb_ref,
               out_ref, sg, sh, st, sc):
    i = pl.program_id(0)
    deg = jnp.sum(deg_ref[...], axis=0)
    norm = 1.0 / jnp.where(deg == 0.0, 1.0, deg)
    a = acc_ref[0] + acc_ref[1]
    hw = jnp.dot(h_ref[...], wself_ref[...],
                 preferred_element_type=jnp.float32)
    repr_blk = jnp.maximum(a * norm[:, None] + hw, 0.0)
    u = jnp.dot(repr_blk, wcat_ref[...],
                preferred_element_type=jnp.float32)
    n_iota = lax.broadcasted_iota(jnp.int32, (B, BN), 1) + i * BN
    b_iota = lax.broadcasted_iota(jnp.int32, (B, BN), 0)
    segm = (seg_ref[...] == b_iota).astype(jnp.float32)
    headm = (head_ref[...] == n_iota).astype(jnp.float32)
    tailm = (tail_ref[...] == n_iota).astype(jnp.float32)
    pg = jnp.dot(segm, u, preferred_element_type=jnp.float32)
    ph = jnp.dot(headm, u, preferred_element_type=jnp.float32)
    pt = jnp.dot(tailm, u, preferred_element_type=jnp.float32)
    cnt = jnp.sum(segm, axis=1, keepdims=True)

    @pl.when(i == 0)
    def _():
        sg[...] = pg
        sh[...] = ph
        st[...] = pt
        sc[...] = cnt

    @pl.when(i > 0)
    def _():
        sg[...] += pg
        sh[...] += ph
        st[...] += pt
        sc[...] += cnt

    @pl.when(i == NB - 1)
    def _():
        r_iota = lax.broadcasted_iota(jnp.int32, (B, R), 1)
        relm = (rlab_ref[...] == r_iota).astype(jnp.float32)
        z = jnp.sum(rel_emb_ref[...] * wr_ref[...], axis=1,
                    keepdims=True)
        rp = jnp.dot(relm, z, preferred_element_type=jnp.float32)
        counts = jnp.maximum(sc[...], 1.0)
        out_ref[...] = sg[:, 0:1] / counts + sh[:, 1:2] + st[:, 2:3] + rp \
            + fcb_ref[...]


_tail = pl.pallas_call(
    _tail_body,
    grid=(NB,),
    in_specs=[
        pl.BlockSpec((NC, BN, D), lambda i: (0, i, 0)),
        pl.BlockSpec((BN, D), lambda i: (i, 0)),
        pl.BlockSpec((D, D), lambda i: (0, 0)),
        pl.BlockSpec((NW, BN), lambda i: (0, i)),
        pl.BlockSpec((1, BN), lambda i: (0, i)),
        pl.BlockSpec((B, 1), lambda i: (0, 0)),
        pl.BlockSpec((B, 1), lambda i: (0, 0)),
        pl.BlockSpec((B, 1), lambda i: (0, 0)),
        pl.BlockSpec((R, RELD), lambda i: (0, 0)),
        pl.BlockSpec((1, RELD), lambda i: (0, 0)),
        pl.BlockSpec((D, 128), lambda i: (0, 0)),
        pl.BlockSpec((1, 1), lambda i: (0, 0)),
    ],
    out_specs=pl.BlockSpec((B, 1), lambda i: (0, 0)),
    out_shape=jax.ShapeDtypeStruct((B, 1), jnp.float32),
    scratch_shapes=[
        pltpu.VMEM((B, 128), jnp.float32),
        pltpu.VMEM((B, 128), jnp.float32),
        pltpu.VMEM((B, 128), jnp.float32),
        pltpu.VMEM((B, 1), jnp.float32),
    ],
)


def kernel(x, edge_index, edge_type, segment_ids, head_ids, tail_ids,
           rel_labels, W_rel, W_self, rel_emb, fc_w, fc_b):
    src = edge_index[0].astype(jnp.int32)
    dst = edge_index[1].astype(jnp.int32)
    et = edge_type.astype(jnp.int32)

    pad_i = jnp.arange(EP - E, dtype=jnp.int32)
    ci = et * NP + src
    ci = jnp.concatenate([ci, pad_i % NP])
    dst_p = jnp.concatenate([dst, N + pad_i % (NP - N)])

    x_p = jnp.pad(x, ((0, NP - N), (0, 0)))
    seg_p = jnp.concatenate(
        [segment_ids.astype(jnp.int32), jnp.full((NP - N,), B, jnp.int32)]
    ).reshape(1, NP)
    deg = _deg(dst_p)
    zeros_init = jnp.broadcast_to(deg[:1, :1] * 0.0, (NP, D))

    h = x_p
    for l in range(L - 1):
        hw = _transform(h, W_rel[l])
        table = hw.reshape(R * NP, D)
        acc = _edge_pass(table, ci.reshape(NW, C, K), dst_p, zeros_init)
        h = _combine(acc.reshape(NC, NP, D), h, W_self[l], deg)

    hw = _transform(h, W_rel[L - 1])
    table = hw.reshape(R * NP, D)
    acc = _edge_pass(table, ci.reshape(NW, C, K), dst_p, zeros_init)

    wcat = jnp.pad(jnp.stack([fc_w[:D, 0], fc_w[D:2 * D, 0],
                              fc_w[2 * D:3 * D, 0]], axis=1),
                   ((0, 0), (0, D - 3)))
    wr = fc_w[3 * D:, 0].reshape(1, RELD)
    out = _tail(acc.reshape(NC, NP, D), h, W_self[L - 1], deg, seg_p,
                head_ids.astype(jnp.int32).reshape(B, 1),
                tail_ids.astype(jnp.int32).reshape(B, 1),
                rel_labels.astype(jnp.int32).reshape(B, 1),
                rel_emb, wr, wcat, fc_b.reshape(1, 1))
    return out


# speedup vs baseline: 1.2179x; 1.0022x over previous
import functools

import jax
import jax.numpy as jnp
from jax import lax
from jax.experimental import pallas as pl
from jax.experimental.pallas import tpu as pltpu
from jax.experimental.pallas import tpu_sc as plsc

N = 10000
E = 320000
D = 128
R = 8
RELD = 32
B = 100
L = 2

NC = 2
NS = 16
NW = NC * NS

NP = 10240
K = 128
C = 80
EPW = C * K
EP = NW * EPW

_MESH = plsc.VectorSubcoreMesh(core_axis_name="c", subcore_axis_name="s")


@functools.partial(
    pl.kernel,
    out_type=jax.ShapeDtypeStruct((NW, NP), jnp.float32),
    mesh=_MESH,
    scratch_types=[
        pltpu.VMEM((EPW,), jnp.int32),
        pltpu.VMEM((NP,), jnp.float32),
    ],
    compiler_params=pltpu.CompilerParams(needs_layout_passes=False),
)
def _deg(dst_hbm, out_hbm, dst_v, deg_v):
    cid = lax.axis_index("c")
    sid = lax.axis_index("s")
    wid = sid * NC + cid
    pltpu.sync_copy(dst_hbm.at[pl.ds(wid * EPW, EPW)], dst_v)

    zeros = jnp.zeros((16,), jnp.float32)
    def zbody(i, _):
        deg_v[pl.ds(pl.multiple_of(i * 16, 16), 16)] = zeros
        return 0
    lax.fori_loop(0, NP // 16, zbody, 0)

    ones = jnp.ones((16,), jnp.float32)
    def body(i, _):
        idx = dst_v[pl.ds(pl.multiple_of(i * 16, 16), 16)]
        plsc.addupdate_scatter(deg_v, [idx], ones)
        return 0
    lax.fori_loop(0, EPW // 16, body, 0)

    pltpu.sync_copy(deg_v, out_hbm.at[wid])


@functools.partial(
    pl.kernel,
    out_type=jax.ShapeDtypeStruct((NC * NP, D), jnp.float32),
    mesh=_MESH,
    scratch_types=[
        pltpu.VMEM((C, K), jnp.int32),
        pltpu.VMEM((K,), jnp.int32),
        pltpu.VMEM((K,), jnp.int32),
        pltpu.VMEM((K, D), jnp.float32),
        pltpu.VMEM((K, D), jnp.float32),
        pltpu.VMEM_SHARED((NP, D), jnp.float32),
        pltpu.SemaphoreType.DMA,
        pltpu.SemaphoreType.DMA,
        pltpu.SemaphoreType.DMA,
        pltpu.SemaphoreType.DMA,
    ],
)
def _edge_pass(table_hbm, ci_hbm, dst_hbm, zeros_hbm, out_hbm,
               ci_v, dst0_v, dst1_v, rows0_v, rows1_v, acc_sh,
               semg0, semg1, semd0, semd1):
    cid = lax.axis_index("c")
    sid = lax.axis_index("s")
    wid = sid * NC + cid
    rpt = NP // NS

    pltpu.sync_copy(zeros_hbm.at[pl.ds(sid * rpt, rpt)],
                    acc_sh.at[pl.ds(sid * rpt, rpt)])
    pltpu.sync_copy(ci_hbm.at[wid], ci_v)
    plsc.subcore_barrier()

    rows = (rows0_v, rows1_v)
    dsts = (dst0_v, dst1_v)
    semg = (semg0, semg1)
    semd = (semd0, semd1)
    base = wid * EPW

    def start(j, b):
        pltpu.async_copy(dst_hbm.at[pl.ds(base + j * K, K)], dsts[b], semd[b])
        pltpu.async_copy(table_hbm.at[ci_v.at[j]], rows[b], semg[b])

    def finish(j, b):
        pltpu.make_async_copy(dst_hbm.at[pl.ds(base + j * K, K)], dsts[b],
                              semd[b]).wait()
        pltpu.make_async_copy(table_hbm.at[ci_v.at[j]], rows[b],
                              semg[b]).wait()
        pltpu.sync_copy(rows[b], acc_sh.at[dsts[b]], add=True)

    start(0, 0)

    def body(i, _):
        jj = i * 2
        for b in range(2):
            j = jj + b

            @pl.when(j + 1 < C)
            def _():
                start(j + 1, 1 - b)

            finish(j, b)
        return 0
    lax.fori_loop(0, C // 2, body, 0)

    plsc.subcore_barrier()
    pltpu.sync_copy(acc_sh.at[pl.ds(sid * rpt, rpt)],
                    out_hbm.at[pl.ds(cid * NP + sid * rpt, rpt)])


BN = 5120
NB = NP // BN


def _transform_body(h_ref, w_ref, out_ref):
    out_ref[0] = jnp.dot(h_ref[...], w_ref[0],
                         preferred_element_type=jnp.float32)


TN = 10240
_transform = pl.pallas_call(
    _transform_body,
    grid=(NP // TN, R),
    in_specs=[
        pl.BlockSpec((TN, D), lambda i, r: (i, 0)),
        pl.BlockSpec((1, D, D), lambda i, r: (r, 0, 0)),
    ],
    out_specs=pl.BlockSpec((1, TN, D), lambda i, r: (r, i, 0)),
    out_shape=jax.ShapeDtypeStruct((R, NP, D), jnp.float32),
)


def _combine_body(acc_ref, h_ref, wself_ref, deg_ref, out_ref):
    deg = jnp.sum(deg_ref[...], axis=0)
    norm = 1.0 / jnp.where(deg == 0.0, 1.0, deg)
    a = acc_ref[0] + acc_ref[1]
    hw = jnp.dot(h_ref[...], wself_ref[...],
                 preferred_element_type=jnp.float32)
    out_ref[...] = jnp.maximum(a * norm[:, None] + hw, 0.0)


_combine = pl.pallas_call(
    _combine_body,
    grid=(NB,),
    in_specs=[
        pl.BlockSpec((NC, BN, D), lambda i: (0, i, 0)),
        pl.BlockSpec((BN, D), lambda i: (i, 0)),
        pl.BlockSpec((D, D), lambda i: (0, 0)),
        pl.BlockSpec((NW, BN), lambda i: (0, i)),
    ],
    out_specs=pl.BlockSpec((BN, D), lambda i: (i, 0)),
    out_shape=jax.ShapeDtypeStruct((NP, D), jnp.float32),
)


def _tail_body(acc_ref, h_ref, wself_ref, deg_ref, seg_ref, head_ref,
               tail_ref, rlab_ref, rel_emb_ref, wr_ref, wcat_ref, fcb_ref,
               out_ref, sg, sh, st, sc):
    i = pl.program_id(0)
    deg = jnp.sum(deg_ref[...], axis=0)
    norm = 1.0 / jnp.where(deg == 0.0, 1.0, deg)
    a = acc_ref[0] + acc_ref[1]
    hw = jnp.dot(h_ref[...], wself_ref[...],
                 preferred_element_type=jnp.float32)
    repr_blk = jnp.maximum(a * norm[:, None] + hw, 0.0)
    u = jnp.dot(repr_blk, wcat_ref[...],
                preferred_element_type=jnp.float32)
    n_iota = lax.broadcasted_iota(jnp.int32, (B, BN), 1) + i * BN
    b_iota = lax.broadcasted_iota(jnp.int32, (B, BN), 0)
    segm = (seg_ref[...] == b_iota).astype(jnp.float32)
    headm = (head_ref[...] == n_iota).astype(jnp.float32)
    tailm = (tail_ref[...] == n_iota).astype(jnp.float32)
    pg = jnp.dot(segm, u, preferred_element_type=jnp.float32)
    ph = jnp.dot(headm, u, preferred_element_type=jnp.float32)
    pt = jnp.dot(tailm, u, preferred_element_type=jnp.float32)
    cnt = jnp.sum(segm, axis=1, keepdims=True)

    @pl.when(i == 0)
    def _():
        sg[...] = pg
        sh[...] = ph
        st[...] = pt
        sc[...] = cnt

    @pl.when(i > 0)
    def _():
        sg[...] += pg
        sh[...] += ph
        st[...] += pt
        sc[...] += cnt

    @pl.when(i == NB - 1)
    def _():
        r_iota = lax.broadcasted_iota(jnp.int32, (B, R), 1)
        relm = (rlab_ref[...] == r_iota).astype(jnp.float32)
        z = jnp.sum(rel_emb_ref[...] * wr_ref[...], axis=1,
                    keepdims=True)
        rp = jnp.dot(relm, z, preferred_element_type=jnp.float32)
        counts = jnp.maximum(sc[...], 1.0)
        out_ref[...] = sg[:, 0:1] / counts + sh[:, 1:2] + st[:, 2:3] + rp \
            + fcb_ref[...]


_tail = pl.pallas_call(
    _tail_body,
    grid=(NB,),
    in_specs=[
        pl.BlockSpec((NC, BN, D), lambda i: (0, i, 0)),
        pl.BlockSpec((BN, D), lambda i: (i, 0)),
        pl.BlockSpec((D, D), lambda i: (0, 0)),
        pl.BlockSpec((NW, BN), lambda i: (0, i)),
        pl.BlockSpec((1, BN), lambda i: (0, i)),
        pl.BlockSpec((B, 1), lambda i: (0, 0)),
        pl.BlockSpec((B, 1), lambda i: (0, 0)),
        pl.BlockSpec((B, 1), lambda i: (0, 0)),
        pl.BlockSpec((R, RELD), lambda i: (0, 0)),
        pl.BlockSpec((1, RELD), lambda i: (0, 0)),
        pl.BlockSpec((D, 128), lambda i: (0, 0)),
        pl.BlockSpec((1, 1), lambda i: (0, 0)),
    ],
    out_specs=pl.BlockSpec((B, 1), lambda i: (0, 0)),
    out_shape=jax.ShapeDtypeStruct((B, 1), jnp.float32),
    scratch_shapes=[
        pltpu.VMEM((B, 128), jnp.float32),
        pltpu.VMEM((B, 128), jnp.float32),
        pltpu.VMEM((B, 128), jnp.float32),
        pltpu.VMEM((B, 1), jnp.float32),
    ],
)


def kernel(x, edge_index, edge_type, segment_ids, head_ids, tail_ids,
           rel_labels, W_rel, W_self, rel_emb, fc_w, fc_b):
    src = edge_index[0].astype(jnp.int32)
    dst = edge_index[1].astype(jnp.int32)
    et = edge_type.astype(jnp.int32)

    pad_i = jnp.arange(EP - E, dtype=jnp.int32)
    ci = et * NP + src
    ci = jnp.concatenate([ci, pad_i % NP])
    dst_p = jnp.concatenate([dst, N + pad_i % (NP - N)])

    x_p = jnp.pad(x, ((0, NP - N), (0, 0)))
    seg_p = jnp.concatenate(
        [segment_ids.astype(jnp.int32), jnp.full((NP - N,), B, jnp.int32)]
    ).reshape(1, NP)
    deg = _deg(dst_p)
    zeros_init = jnp.broadcast_to(deg[:1, :1] * 0.0, (NP, D))

    h = x_p
    for l in range(L - 1):
        hw = _transform(h, W_rel[l])
        table = hw.reshape(R * NP, D)
        acc = _edge_pass(table, ci.reshape(NW, C, K), dst_p, zeros_init)
        h = _combine(acc.reshape(NC, NP, D), h, W_self[l], deg)

    hw = _transform(h, W_rel[L - 1])
    table = hw.reshape(R * NP, D)
    acc = _edge_pass(table, ci.reshape(NW, C, K), dst_p, zeros_init)

    wcat = jnp.pad(jnp.stack([fc_w[:D, 0], fc_w[D:2 * D, 0],
                              fc_w[2 * D:3 * D, 0]], axis=1),
                   ((0, 0), (0, D - 3)))
    wr = fc_w[3 * D:, 0].reshape(1, RELD)
    out = _tail(acc.reshape(NC, NP, D), h, W_self[L - 1], deg, seg_p,
                head_ids.astype(jnp.int32).reshape(B, 1),
                tail_ids.astype(jnp.int32).reshape(B, 1),
                rel_labels.astype(jnp.int32).reshape(B, 1),
                rel_emb, wr, wcat, fc_b.reshape(1, 1))
    return out


# speedup vs baseline: 1.2263x; 1.0069x over previous
import functools

import jax
import jax.numpy as jnp
from jax import lax
from jax.experimental import pallas as pl
from jax.experimental.pallas import tpu as pltpu
from jax.experimental.pallas import tpu_sc as plsc

N = 10000
E = 320000
D = 128
R = 8
RELD = 32
B = 100
L = 2

NC = 2
NS = 16
NW = NC * NS

NP = 10240
K = 128
C = 80
EPW = C * K
EP = NW * EPW

_MESH = plsc.VectorSubcoreMesh(core_axis_name="c", subcore_axis_name="s")


@functools.partial(
    pl.kernel,
    out_type=jax.ShapeDtypeStruct((NW, NP), jnp.float32),
    mesh=_MESH,
    scratch_types=[
        pltpu.VMEM((EPW,), jnp.int32),
        pltpu.VMEM((NP,), jnp.float32),
    ],
    compiler_params=pltpu.CompilerParams(needs_layout_passes=False),
)
def _deg(dst_hbm, out_hbm, dst_v, deg_v):
    cid = lax.axis_index("c")
    sid = lax.axis_index("s")
    wid = sid * NC + cid
    pltpu.sync_copy(dst_hbm.at[pl.ds(wid * EPW, EPW)], dst_v)

    zeros = jnp.zeros((16,), jnp.float32)
    def zbody(i, _):
        deg_v[pl.ds(pl.multiple_of(i * 16, 16), 16)] = zeros
        return 0
    lax.fori_loop(0, NP // 16, zbody, 0)

    ones = jnp.ones((16,), jnp.float32)
    def body(i, _):
        idx = dst_v[pl.ds(pl.multiple_of(i * 16, 16), 16)]
        plsc.addupdate_scatter(deg_v, [idx], ones)
        return 0
    lax.fori_loop(0, EPW // 16, body, 0)

    pltpu.sync_copy(deg_v, out_hbm.at[wid])


@functools.partial(
    pl.kernel,
    out_type=jax.ShapeDtypeStruct((NC * NP, D), jnp.float32),
    mesh=_MESH,
    scratch_types=[
        pltpu.VMEM((C, K), jnp.int32),
        pltpu.VMEM((K,), jnp.int32),
        pltpu.VMEM((K,), jnp.int32),
        pltpu.VMEM((K, D), jnp.float32),
        pltpu.VMEM((K, D), jnp.float32),
        pltpu.VMEM_SHARED((NP, D), jnp.float32),
        pltpu.SemaphoreType.DMA,
        pltpu.SemaphoreType.DMA,
        pltpu.SemaphoreType.DMA,
        pltpu.SemaphoreType.DMA,
    ],
)
def _edge_pass(table_hbm, ci_hbm, dst_hbm, zeros_hbm, deg_hbm, out_hbm,
               ci_v, dst0_v, dst1_v, rows0_v, rows1_v, acc_sh,
               semg0, semg1, semd0, semd1):
    del deg_hbm
    cid = lax.axis_index("c")
    sid = lax.axis_index("s")
    wid = sid * NC + cid
    rpt = NP // NS

    pltpu.sync_copy(zeros_hbm.at[pl.ds(sid * rpt, rpt)],
                    acc_sh.at[pl.ds(sid * rpt, rpt)])
    pltpu.sync_copy(ci_hbm.at[wid], ci_v)
    plsc.subcore_barrier()

    rows = (rows0_v, rows1_v)
    dsts = (dst0_v, dst1_v)
    semg = (semg0, semg1)
    semd = (semd0, semd1)
    base = wid * EPW

    def start(j, b):
        pltpu.async_copy(dst_hbm.at[pl.ds(base + j * K, K)], dsts[b], semd[b])
        pltpu.async_copy(table_hbm.at[ci_v.at[j]], rows[b], semg[b])

    def finish(j, b):
        pltpu.make_async_copy(dst_hbm.at[pl.ds(base + j * K, K)], dsts[b],
                              semd[b]).wait()
        pltpu.make_async_copy(table_hbm.at[ci_v.at[j]], rows[b],
                              semg[b]).wait()
        pltpu.sync_copy(rows[b], acc_sh.at[dsts[b]], add=True)

    start(0, 0)

    def body(i, _):
        jj = i * 2
        for b in range(2):
            j = jj + b

            @pl.when(j + 1 < C)
            def _():
                start(j + 1, 1 - b)

            finish(j, b)
        return 0
    lax.fori_loop(0, C // 2, body, 0)

    plsc.subcore_barrier()
    pltpu.sync_copy(acc_sh.at[pl.ds(sid * rpt, rpt)],
                    out_hbm.at[pl.ds(cid * NP + sid * rpt, rpt)])


BN = 5120
NB = NP // BN


def _transform_body(h_ref, w_ref, out_ref):
    out_ref[0] = jnp.dot(h_ref[...], w_ref[0],
                         preferred_element_type=jnp.float32)


TN = 10240
_transform = pl.pallas_call(
    _transform_body,
    grid=(NP // TN, R),
    in_specs=[
        pl.BlockSpec((TN, D), lambda i, r: (i, 0)),
        pl.BlockSpec((1, D, D), lambda i, r: (r, 0, 0)),
    ],
    out_specs=pl.BlockSpec((1, TN, D), lambda i, r: (r, i, 0)),
    out_shape=jax.ShapeDtypeStruct((R, NP, D), jnp.float32),
)


def _combine_body(acc_ref, h_ref, wself_ref, deg_ref, out_ref):
    deg = jnp.sum(deg_ref[...], axis=0)
    norm = 1.0 / jnp.where(deg == 0.0, 1.0, deg)
    a = acc_ref[0] + acc_ref[1]
    hw = jnp.dot(h_ref[...], wself_ref[...],
                 preferred_element_type=jnp.float32)
    out_ref[...] = jnp.maximum(a * norm[:, None] + hw, 0.0)


_combine = pl.pallas_call(
    _combine_body,
    grid=(NB,),
    in_specs=[
        pl.BlockSpec((NC, BN, D), lambda i: (0, i, 0)),
        pl.BlockSpec((BN, D), lambda i: (i, 0)),
        pl.BlockSpec((D, D), lambda i: (0, 0)),
        pl.BlockSpec((NW, BN), lambda i: (0, i)),
    ],
    out_specs=pl.BlockSpec((BN, D), lambda i: (i, 0)),
    out_shape=jax.ShapeDtypeStruct((NP, D), jnp.float32),
)


def _tail_body(acc_ref, h_ref, wself_ref, deg_ref, seg_ref, head_ref,
               tail_ref, rlab_ref, rel_emb_ref, wr_ref, wcat_ref, fcb_ref,
               out_ref, sg, sh, st, sc):
    i = pl.program_id(0)
    deg = jnp.sum(deg_ref[...], axis=0)
    norm = 1.0 / jnp.where(deg == 0.0, 1.0, deg)
    a = acc_ref[0] + acc_ref[1]
    hw = jnp.dot(h_ref[...], wself_ref[...],
                 preferred_element_type=jnp.float32)
    repr_blk = jnp.maximum(a * norm[:, None] + hw, 0.0)
    u = jnp.dot(repr_blk, wcat_ref[...],
                preferred_element_type=jnp.float32)
    n_iota = lax.broadcasted_iota(jnp.int32, (B, BN), 1) + i * BN
    b_iota = lax.broadcasted_iota(jnp.int32, (B, BN), 0)
    segm = (seg_ref[...] == b_iota).astype(jnp.float32)
    headm = (head_ref[...] == n_iota).astype(jnp.float32)
    tailm = (tail_ref[...] == n_iota).astype(jnp.float32)
    pg = jnp.dot(segm, u, preferred_element_type=jnp.float32)
    ph = jnp.dot(headm, u, preferred_element_type=jnp.float32)
    pt = jnp.dot(tailm, u, preferred_element_type=jnp.float32)
    cnt = jnp.sum(segm, axis=1, keepdims=True)

    @pl.when(i == 0)
    def _():
        sg[...] = pg
        sh[...] = ph
        st[...] = pt
        sc[...] = cnt

    @pl.when(i > 0)
    def _():
        sg[...] += pg
        sh[...] += ph
        st[...] += pt
        sc[...] += cnt

    @pl.when(i == NB - 1)
    def _():
        r_iota = lax.broadcasted_iota(jnp.int32, (B, R), 1)
        relm = (rlab_ref[...] == r_iota).astype(jnp.float32)
        z = jnp.sum(rel_emb_ref[...] * wr_ref[...], axis=1,
                    keepdims=True)
        rp = jnp.dot(relm, z, preferred_element_type=jnp.float32)
        counts = jnp.maximum(sc[...], 1.0)
        out_ref[...] = sg[:, 0:1] / counts + sh[:, 1:2] + st[:, 2:3] + rp \
            + fcb_ref[...]


_tail = pl.pallas_call(
    _tail_body,
    grid=(NB,),
    in_specs=[
        pl.BlockSpec((NC, BN, D), lambda i: (0, i, 0)),
        pl.BlockSpec((BN, D), lambda i: (i, 0)),
        pl.BlockSpec((D, D), lambda i: (0, 0)),
        pl.BlockSpec((NW, BN), lambda i: (0, i)),
        pl.BlockSpec((1, BN), lambda i: (0, i)),
        pl.BlockSpec((B, 1), lambda i: (0, 0)),
        pl.BlockSpec((B, 1), lambda i: (0, 0)),
        pl.BlockSpec((B, 1), lambda i: (0, 0)),
        pl.BlockSpec((R, RELD), lambda i: (0, 0)),
        pl.BlockSpec((1, RELD), lambda i: (0, 0)),
        pl.BlockSpec((D, 128), lambda i: (0, 0)),
        pl.BlockSpec((1, 1), lambda i: (0, 0)),
    ],
    out_specs=pl.BlockSpec((B, 1), lambda i: (0, 0)),
    out_shape=jax.ShapeDtypeStruct((B, 1), jnp.float32),
    scratch_shapes=[
        pltpu.VMEM((B, 128), jnp.float32),
        pltpu.VMEM((B, 128), jnp.float32),
        pltpu.VMEM((B, 128), jnp.float32),
        pltpu.VMEM((B, 1), jnp.float32),
    ],
)


def kernel(x, edge_index, edge_type, segment_ids, head_ids, tail_ids,
           rel_labels, W_rel, W_self, rel_emb, fc_w, fc_b):
    src = edge_index[0].astype(jnp.int32)
    dst = edge_index[1].astype(jnp.int32)
    et = edge_type.astype(jnp.int32)

    pad_i = jnp.arange(EP - E, dtype=jnp.int32)
    ci = et * NP + src
    ci = jnp.concatenate([ci, pad_i % NP])
    dst_p = jnp.concatenate([dst, N + pad_i % (NP - N)])

    x_p = jnp.pad(x, ((0, NP - N), (0, 0)))
    seg_p = jnp.concatenate(
        [segment_ids.astype(jnp.int32), jnp.full((NP - N,), B, jnp.int32)]
    ).reshape(1, NP)
    deg = _deg(dst_p)
    zeros_init = jnp.zeros((NP, D), jnp.float32)

    h = x_p
    for l in range(L - 1):
        hw = _transform(h, W_rel[l])
        table = hw.reshape(R * NP, D)
        acc = _edge_pass(table, ci.reshape(NW, C, K), dst_p, zeros_init, deg)
        h = _combine(acc.reshape(NC, NP, D), h, W_self[l], deg)

    hw = _transform(h, W_rel[L - 1])
    table = hw.reshape(R * NP, D)
    acc = _edge_pass(table, ci.reshape(NW, C, K), dst_p, zeros_init, deg)

    wcat = jnp.pad(jnp.stack([fc_w[:D, 0], fc_w[D:2 * D, 0],
                              fc_w[2 * D:3 * D, 0]], axis=1),
                   ((0, 0), (0, D - 3)))
    wr = fc_w[3 * D:, 0].reshape(1, RELD)
    out = _tail(acc.reshape(NC, NP, D), h, W_self[L - 1], deg, seg_p,
                head_ids.astype(jnp.int32).reshape(B, 1),
                tail_ids.astype(jnp.int32).reshape(B, 1),
                rel_labels.astype(jnp.int32).reshape(B, 1),
                rel_emb, wr, wcat, fc_b.reshape(1, 1))
    return out


# speedup vs baseline: 1.2486x; 1.0182x over previous
import functools

import jax
import jax.numpy as jnp
from jax import lax
from jax.experimental import pallas as pl
from jax.experimental.pallas import tpu as pltpu
from jax.experimental.pallas import tpu_sc as plsc

N = 10000
E = 320000
D = 128
R = 8
RELD = 32
B = 100
L = 2

NC = 2
NS = 16
NW = NC * NS

NP = 10240
K = 128
C = 80
EPW = C * K
EP = NW * EPW

_MESH = plsc.VectorSubcoreMesh(core_axis_name="c", subcore_axis_name="s")


@functools.partial(
    pl.kernel,
    out_type=jax.ShapeDtypeStruct((NW, NP), jnp.float32),
    mesh=_MESH,
    scratch_types=[
        pltpu.VMEM((EPW,), jnp.int32),
        pltpu.VMEM((NP,), jnp.float32),
    ],
    compiler_params=pltpu.CompilerParams(needs_layout_passes=False),
)
def _deg(dst_hbm, out_hbm, dst_v, deg_v):
    cid = lax.axis_index("c")
    sid = lax.axis_index("s")
    wid = sid * NC + cid
    pltpu.sync_copy(dst_hbm.at[pl.ds(wid * EPW, EPW)], dst_v)

    zeros = jnp.zeros((16,), jnp.float32)
    def zbody(i, _):
        deg_v[pl.ds(pl.multiple_of(i * 16, 16), 16)] = zeros
        return 0
    lax.fori_loop(0, NP // 16, zbody, 0)

    ones = jnp.ones((16,), jnp.float32)
    def body(i, _):
        idx = dst_v[pl.ds(pl.multiple_of(i * 16, 16), 16)]
        plsc.addupdate_scatter(deg_v, [idx], ones)
        return 0
    lax.fori_loop(0, EPW // 16, body, 0)

    pltpu.sync_copy(deg_v, out_hbm.at[wid])


@functools.partial(
    pl.kernel,
    out_type=jax.ShapeDtypeStruct((NC * NP, D), jnp.float32),
    mesh=_MESH,
    scratch_types=[
        pltpu.VMEM((C, K), jnp.int32),
        pltpu.VMEM((K,), jnp.int32),
        pltpu.VMEM((K,), jnp.int32),
        pltpu.VMEM((K, D), jnp.float32),
        pltpu.VMEM((K, D), jnp.float32),
        pltpu.VMEM_SHARED((NP, D), jnp.float32),
        pltpu.SemaphoreType.DMA,
        pltpu.SemaphoreType.DMA,
        pltpu.SemaphoreType.DMA,
        pltpu.SemaphoreType.DMA,
        pltpu.SemaphoreType.DMA,
    ],
)
def _edge_pass(table_hbm, ci_hbm, dst_hbm, zeros_hbm, deg_hbm, out_hbm,
               ci_v, dst0_v, dst1_v, rows0_v, rows1_v, acc_sh,
               semg0, semg1, semd0, semd1, semz):
    del deg_hbm
    cid = lax.axis_index("c")
    sid = lax.axis_index("s")
    wid = sid * NC + cid
    rpt = NP // NS

    pltpu.async_copy(zeros_hbm.at[pl.ds(sid * rpt, rpt)],
                     acc_sh.at[pl.ds(sid * rpt, rpt)], semz)
    pltpu.sync_copy(ci_hbm.at[wid], ci_v)

    rows = (rows0_v, rows1_v)
    dsts = (dst0_v, dst1_v)
    semg = (semg0, semg1)
    semd = (semd0, semd1)
    base = wid * EPW

    def start(j, b):
        pltpu.async_copy(dst_hbm.at[pl.ds(base + j * K, K)], dsts[b], semd[b])
        pltpu.async_copy(table_hbm.at[ci_v.at[j]], rows[b], semg[b])

    def finish(j, b):
        pltpu.make_async_copy(dst_hbm.at[pl.ds(base + j * K, K)], dsts[b],
                              semd[b]).wait()
        pltpu.make_async_copy(table_hbm.at[ci_v.at[j]], rows[b],
                              semg[b]).wait()
        pltpu.sync_copy(rows[b], acc_sh.at[dsts[b]], add=True)

    start(0, 0)
    pltpu.make_async_copy(zeros_hbm.at[pl.ds(sid * rpt, rpt)],
                          acc_sh.at[pl.ds(sid * rpt, rpt)], semz).wait()
    plsc.subcore_barrier()

    def body(i, _):
        jj = i * 2
        for b in range(2):
            j = jj + b

            @pl.when(j + 1 < C)
            def _():
                start(j + 1, 1 - b)

            finish(j, b)
        return 0
    lax.fori_loop(0, C // 2, body, 0)

    plsc.subcore_barrier()
    pltpu.sync_copy(acc_sh.at[pl.ds(sid * rpt, rpt)],
                    out_hbm.at[pl.ds(cid * NP + sid * rpt, rpt)])


BN = 5120
NB = NP // BN


def _transform_body(h_ref, w_ref, out_ref):
    out_ref[0] = jnp.dot(h_ref[...], w_ref[0],
                         preferred_element_type=jnp.float32)


TN = 10240
_transform = pl.pallas_call(
    _transform_body,
    grid=(NP // TN, R),
    in_specs=[
        pl.BlockSpec((TN, D), lambda i, r: (i, 0)),
        pl.BlockSpec((1, D, D), lambda i, r: (r, 0, 0)),
    ],
    out_specs=pl.BlockSpec((1, TN, D), lambda i, r: (r, i, 0)),
    out_shape=jax.ShapeDtypeStruct((R, NP, D), jnp.float32),
)


def _combine_body(acc_ref, h_ref, wself_ref, deg_ref, out_ref):
    deg = jnp.sum(deg_ref[...], axis=0)
    norm = 1.0 / jnp.where(deg == 0.0, 1.0, deg)
    a = acc_ref[0] + acc_ref[1]
    hw = jnp.dot(h_ref[...], wself_ref[...],
                 preferred_element_type=jnp.float32)
    out_ref[...] = jnp.maximum(a * norm[:, None] + hw, 0.0)


_combine = pl.pallas_call(
    _combine_body,
    grid=(NB,),
    in_specs=[
        pl.BlockSpec((NC, BN, D), lambda i: (0, i, 0)),
        pl.BlockSpec((BN, D), lambda i: (i, 0)),
        pl.BlockSpec((D, D), lambda i: (0, 0)),
        pl.BlockSpec((NW, BN), lambda i: (0, i)),
    ],
    out_specs=pl.BlockSpec((BN, D), lambda i: (i, 0)),
    out_shape=jax.ShapeDtypeStruct((NP, D), jnp.float32),
)


def _tail_body(acc_ref, h_ref, wself_ref, deg_ref, seg_ref, head_ref,
               tail_ref, rlab_ref, rel_emb_ref, wr_ref, wcat_ref, fcb_ref,
               out_ref, sg, sh, st, sc):
    i = pl.program_id(0)
    deg = jnp.sum(deg_ref[...], axis=0)
    norm = 1.0 / jnp.where(deg == 0.0, 1.0, deg)
    a = acc_ref[0] + acc_ref[1]
    hw = jnp.dot(h_ref[...], wself_ref[...],
                 preferred_element_type=jnp.float32)
    repr_blk = jnp.maximum(a * norm[:, None] + hw, 0.0)
    u = jnp.dot(repr_blk, wcat_ref[...],
                preferred_element_type=jnp.float32)
    n_iota = lax.broadcasted_iota(jnp.int32, (B, BN), 1) + i * BN
    b_iota = lax.broadcasted_iota(jnp.int32, (B, BN), 0)
    segm = (seg_ref[...] == b_iota).astype(jnp.float32)
    headm = (head_ref[...] == n_iota).astype(jnp.float32)
    tailm = (tail_ref[...] == n_iota).astype(jnp.float32)
    pg = jnp.dot(segm, u, preferred_element_type=jnp.float32)
    ph = jnp.dot(headm, u, preferred_element_type=jnp.float32)
    pt = jnp.dot(tailm, u, preferred_element_type=jnp.float32)
    cnt = jnp.sum(segm, axis=1, keepdims=True)

    @pl.when(i == 0)
    def _():
        sg[...] = pg
        sh[...] = ph
        st[...] = pt
        sc[...] = cnt

    @pl.when(i > 0)
    def _():
        sg[...] += pg
        sh[...] += ph
        st[...] += pt
        sc[...] += cnt

    @pl.when(i == NB - 1)
    def _():
        r_iota = lax.broadcasted_iota(jnp.int32, (B, R), 1)
        relm = (rlab_ref[...] == r_iota).astype(jnp.float32)
        z = jnp.sum(rel_emb_ref[...] * wr_ref[...], axis=1,
                    keepdims=True)
        rp = jnp.dot(relm, z, preferred_element_type=jnp.float32)
        counts = jnp.maximum(sc[...], 1.0)
        out_ref[...] = sg[:, 0:1] / counts + sh[:, 1:2] + st[:, 2:3] + rp \
            + fcb_ref[...]


_tail = pl.pallas_call(
    _tail_body,
    grid=(NB,),
    in_specs=[
        pl.BlockSpec((NC, BN, D), lambda i: (0, i, 0)),
        pl.BlockSpec((BN, D), lambda i: (i, 0)),
        pl.BlockSpec((D, D), lambda i: (0, 0)),
        pl.BlockSpec((NW, BN), lambda i: (0, i)),
        pl.BlockSpec((1, BN), lambda i: (0, i)),
        pl.BlockSpec((B, 1), lambda i: (0, 0)),
        pl.BlockSpec((B, 1), lambda i: (0, 0)),
        pl.BlockSpec((B, 1), lambda i: (0, 0)),
        pl.BlockSpec((R, RELD), lambda i: (0, 0)),
        pl.BlockSpec((1, RELD), lambda i: (0, 0)),
        pl.BlockSpec((D, 128), lambda i: (0, 0)),
        pl.BlockSpec((1, 1), lambda i: (0, 0)),
    ],
    out_specs=pl.BlockSpec((B, 1), lambda i: (0, 0)),
    out_shape=jax.ShapeDtypeStruct((B, 1), jnp.float32),
    scratch_shapes=[
        pltpu.VMEM((B, 128), jnp.float32),
        pltpu.VMEM((B, 128), jnp.float32),
        pltpu.VMEM((B, 128), jnp.float32),
        pltpu.VMEM((B, 1), jnp.float32),
    ],
)


def kernel(x, edge_index, edge_type, segment_ids, head_ids, tail_ids,
           rel_labels, W_rel, W_self, rel_emb, fc_w, fc_b):
    src = edge_index[0].astype(jnp.int32)
    dst = edge_index[1].astype(jnp.int32)
    et = edge_type.astype(jnp.int32)

    pad_i = jnp.arange(EP - E, dtype=jnp.int32)
    ci = et * NP + src
    ci = jnp.concatenate([ci, pad_i % NP])
    dst_p = jnp.concatenate([dst, N + pad_i % (NP - N)])

    x_p = jnp.pad(x, ((0, NP - N), (0, 0)))
    seg_p = jnp.concatenate(
        [segment_ids.astype(jnp.int32), jnp.full((NP - N,), B, jnp.int32)]
    ).reshape(1, NP)
    deg = _deg(dst_p)
    zeros_init = jnp.zeros((NP, D), jnp.float32)

    h = x_p
    for l in range(L - 1):
        hw = _transform(h, W_rel[l])
        table = hw.reshape(R * NP, D)
        acc = _edge_pass(table, ci.reshape(NW, C, K), dst_p, zeros_init, deg)
        h = _combine(acc.reshape(NC, NP, D), h, W_self[l], deg)

    hw = _transform(h, W_rel[L - 1])
    table = hw.reshape(R * NP, D)
    acc = _edge_pass(table, ci.reshape(NW, C, K), dst_p, zeros_init, deg)

    wcat = jnp.pad(jnp.stack([fc_w[:D, 0], fc_w[D:2 * D, 0],
                              fc_w[2 * D:3 * D, 0]], axis=1),
                   ((0, 0), (0, D - 3)))
    wr = fc_w[3 * D:, 0].reshape(1, RELD)
    out = _tail(acc.reshape(NC, NP, D), h, W_self[L - 1], deg, seg_p,
                head_ids.astype(jnp.int32).reshape(B, 1),
                tail_ids.astype(jnp.int32).reshape(B, 1),
                rel_labels.astype(jnp.int32).reshape(B, 1),
                rel_emb, wr, wcat, fc_b.reshape(1, 1))
    return out


# speedup vs baseline: 1.2544x; 1.0047x over previous
import functools

import jax
import jax.numpy as jnp
from jax import lax
from jax.experimental import pallas as pl
from jax.experimental.pallas import tpu as pltpu
from jax.experimental.pallas import tpu_sc as plsc

N = 10000
E = 320000
D = 128
R = 8
RELD = 32
B = 100
L = 2

NC = 2
NS = 16
NW = NC * NS

NP = 10240
K = 128
C = 80
EPW = C * K
EP = NW * EPW

_MESH = plsc.VectorSubcoreMesh(core_axis_name="c", subcore_axis_name="s")


@functools.partial(
    pl.kernel,
    out_type=jax.ShapeDtypeStruct((NW, NP), jnp.float32),
    mesh=_MESH,
    scratch_types=[
        pltpu.VMEM((EPW,), jnp.int32),
        pltpu.VMEM((NP,), jnp.float32),
    ],
    compiler_params=pltpu.CompilerParams(needs_layout_passes=False),
)
def _deg(dst_hbm, out_hbm, dst_v, deg_v):
    cid = lax.axis_index("c")
    sid = lax.axis_index("s")
    wid = sid * NC + cid
    pltpu.sync_copy(dst_hbm.at[pl.ds(wid * EPW, EPW)], dst_v)

    zeros = jnp.zeros((16,), jnp.float32)
    def zbody(i, _):
        deg_v[pl.ds(pl.multiple_of(i * 16, 16), 16)] = zeros
        return 0
    lax.fori_loop(0, NP // 16, zbody, 0)

    ones = jnp.ones((16,), jnp.float32)
    def body(i, _):
        idx = dst_v[pl.ds(pl.multiple_of(i * 16, 16), 16)]
        plsc.addupdate_scatter(deg_v, [idx], ones)
        return 0
    lax.fori_loop(0, EPW // 16, body, 0)

    pltpu.sync_copy(deg_v, out_hbm.at[wid])


@functools.partial(
    pl.kernel,
    out_type=jax.ShapeDtypeStruct((NC * NP, D), jnp.float32),
    mesh=_MESH,
    scratch_types=[
        pltpu.VMEM((C, K), jnp.int32),
        pltpu.VMEM((K,), jnp.int32),
        pltpu.VMEM((K,), jnp.int32),
        pltpu.VMEM((K, D), jnp.float32),
        pltpu.VMEM((K, D), jnp.float32),
        pltpu.VMEM_SHARED((NP, D), jnp.float32),
        pltpu.SemaphoreType.DMA,
        pltpu.SemaphoreType.DMA,
        pltpu.SemaphoreType.DMA,
        pltpu.SemaphoreType.DMA,
        pltpu.SemaphoreType.DMA,
    ],
)
def _edge_pass(table_hbm, ci_hbm, dst_hbm, zeros_hbm, deg_hbm, out_hbm,
               ci_v, dst0_v, dst1_v, rows0_v, rows1_v, acc_sh,
               semg0, semg1, semd0, semd1, semz):
    del deg_hbm
    cid = lax.axis_index("c")
    sid = lax.axis_index("s")
    wid = sid * NC + cid
    rpt = NP // NS

    pltpu.async_copy(zeros_hbm.at[pl.ds(sid * rpt, rpt)],
                     acc_sh.at[pl.ds(sid * rpt, rpt)], semz)
    pltpu.sync_copy(ci_hbm.at[wid], ci_v)

    rows = (rows0_v, rows1_v)
    dsts = (dst0_v, dst1_v)
    semg = (semg0, semg1)
    semd = (semd0, semd1)
    base = wid * EPW

    def start(j, b):
        pltpu.async_copy(dst_hbm.at[pl.ds(base + j * K, K)], dsts[b], semd[b])
        pltpu.async_copy(table_hbm.at[ci_v.at[j]], rows[b], semg[b])

    def finish(j, b):
        pltpu.make_async_copy(dst_hbm.at[pl.ds(base + j * K, K)], dsts[b],
                              semd[b]).wait()
        pltpu.make_async_copy(table_hbm.at[ci_v.at[j]], rows[b],
                              semg[b]).wait()
        pltpu.sync_copy(rows[b], acc_sh.at[dsts[b]], add=True)

    start(0, 0)
    pltpu.make_async_copy(zeros_hbm.at[pl.ds(sid * rpt, rpt)],
                          acc_sh.at[pl.ds(sid * rpt, rpt)], semz).wait()
    plsc.subcore_barrier()

    def body(i, _):
        jj = i * 2
        for b in range(2):
            j = jj + b

            @pl.when(j + 1 < C)
            def _():
                start(j + 1, 1 - b)

            finish(j, b)
        return 0
    lax.fori_loop(0, C // 2, body, 0)

    plsc.subcore_barrier()
    pltpu.sync_copy(acc_sh.at[pl.ds(sid * rpt, rpt)],
                    out_hbm.at[pl.ds(cid * NP + sid * rpt, rpt)])


BN = 5120
NB = NP // BN


def _transform_body(h_ref, w_ref, out_ref):
    out_ref[0] = jnp.dot(h_ref[...], w_ref[0],
                         preferred_element_type=jnp.float32)


TN = 10240
_transform = pl.pallas_call(
    _transform_body,
    grid=(NP // TN, R),
    in_specs=[
        pl.BlockSpec((TN, D), lambda i, r: (i, 0)),
        pl.BlockSpec((1, D, D), lambda i, r: (r, 0, 0)),
    ],
    out_specs=pl.BlockSpec((1, TN, D), lambda i, r: (r, i, 0)),
    out_shape=jax.ShapeDtypeStruct((R, NP, D), jnp.float32),
)


def _combine_transform_body(acc_ref, h_ref, wself_ref, deg_ref, wrel_ref,
                            hw_ref, h1_ref):
    r = pl.program_id(0)

    @pl.when(r == 0)
    def _():
        deg = jnp.sum(deg_ref[...], axis=0)
        norm = 1.0 / jnp.where(deg == 0.0, 1.0, deg)
        a = acc_ref[0] + acc_ref[1]
        hw = jnp.dot(h_ref[...], wself_ref[...],
                     preferred_element_type=jnp.float32)
        h1_ref[...] = jnp.maximum(a * norm[:, None] + hw, 0.0)

    hw_ref[0] = jnp.dot(h1_ref[...], wrel_ref[0],
                        preferred_element_type=jnp.float32)


_combine_transform = pl.pallas_call(
    _combine_transform_body,
    grid=(R,),
    in_specs=[
        pl.BlockSpec((NC, NP, D), lambda r: (0, 0, 0)),
        pl.BlockSpec((NP, D), lambda r: (0, 0)),
        pl.BlockSpec((D, D), lambda r: (0, 0)),
        pl.BlockSpec((NW, NP), lambda r: (0, 0)),
        pl.BlockSpec((1, D, D), lambda r: (r, 0, 0)),
    ],
    out_specs=[
        pl.BlockSpec((1, NP, D), lambda r: (r, 0, 0)),
        pl.BlockSpec((NP, D), lambda r: (0, 0)),
    ],
    out_shape=[
        jax.ShapeDtypeStruct((R, NP, D), jnp.float32),
        jax.ShapeDtypeStruct((NP, D), jnp.float32),
    ],
)


def _tail_body(acc_ref, h_ref, wself_ref, deg_ref, seg_ref, head_ref,
               tail_ref, rlab_ref, rel_emb_ref, wr_ref, wcat_ref, fcb_ref,
               out_ref, sg, sh, st, sc):
    i = pl.program_id(0)
    deg = jnp.sum(deg_ref[...], axis=0)
    norm = 1.0 / jnp.where(deg == 0.0, 1.0, deg)
    a = acc_ref[0] + acc_ref[1]
    hw = jnp.dot(h_ref[...], wself_ref[...],
                 preferred_element_type=jnp.float32)
    repr_blk = jnp.maximum(a * norm[:, None] + hw, 0.0)
    u = jnp.dot(repr_blk, wcat_ref[...],
                preferred_element_type=jnp.float32)
    n_iota = lax.broadcasted_iota(jnp.int32, (B, BN), 1) + i * BN
    b_iota = lax.broadcasted_iota(jnp.int32, (B, BN), 0)
    segm = (seg_ref[...] == b_iota).astype(jnp.float32)
    headm = (head_ref[...] == n_iota).astype(jnp.float32)
    tailm = (tail_ref[...] == n_iota).astype(jnp.float32)
    pg = jnp.dot(segm, u, preferred_element_type=jnp.float32)
    ph = jnp.dot(headm, u, preferred_element_type=jnp.float32)
    pt = jnp.dot(tailm, u, preferred_element_type=jnp.float32)
    cnt = jnp.sum(segm, axis=1, keepdims=True)

    @pl.when(i == 0)
    def _():
        sg[...] = pg
        sh[...] = ph
        st[...] = pt
        sc[...] = cnt

    @pl.when(i > 0)
    def _():
        sg[...] += pg
        sh[...] += ph
        st[...] += pt
        sc[...] += cnt

    @pl.when(i == NB - 1)
    def _():
        r_iota = lax.broadcasted_iota(jnp.int32, (B, R), 1)
        relm = (rlab_ref[...] == r_iota).astype(jnp.float32)
        z = jnp.sum(rel_emb_ref[...] * wr_ref[...], axis=1,
                    keepdims=True)
        rp = jnp.dot(relm, z, preferred_element_type=jnp.float32)
        counts = jnp.maximum(sc[...], 1.0)
        out_ref[...] = sg[:, 0:1] / counts + sh[:, 1:2] + st[:, 2:3] + rp \
            + fcb_ref[...]


_tail = pl.pallas_call(
    _tail_body,
    grid=(NB,),
    in_specs=[
        pl.BlockSpec((NC, BN, D), lambda i: (0, i, 0)),
        pl.BlockSpec((BN, D), lambda i: (i, 0)),
        pl.BlockSpec((D, D), lambda i: (0, 0)),
        pl.BlockSpec((NW, BN), lambda i: (0, i)),
        pl.BlockSpec((1, BN), lambda i: (0, i)),
        pl.BlockSpec((B, 1), lambda i: (0, 0)),
        pl.BlockSpec((B, 1), lambda i: (0, 0)),
        pl.BlockSpec((B, 1), lambda i: (0, 0)),
        pl.BlockSpec((R, RELD), lambda i: (0, 0)),
        pl.BlockSpec((1, RELD), lambda i: (0, 0)),
        pl.BlockSpec((D, 128), lambda i: (0, 0)),
        pl.BlockSpec((1, 1), lambda i: (0, 0)),
    ],
    out_specs=pl.BlockSpec((B, 1), lambda i: (0, 0)),
    out_shape=jax.ShapeDtypeStruct((B, 1), jnp.float32),
    scratch_shapes=[
        pltpu.VMEM((B, 128), jnp.float32),
        pltpu.VMEM((B, 128), jnp.float32),
        pltpu.VMEM((B, 128), jnp.float32),
        pltpu.VMEM((B, 1), jnp.float32),
    ],
)


def kernel(x, edge_index, edge_type, segment_ids, head_ids, tail_ids,
           rel_labels, W_rel, W_self, rel_emb, fc_w, fc_b):
    src = edge_index[0].astype(jnp.int32)
    dst = edge_index[1].astype(jnp.int32)
    et = edge_type.astype(jnp.int32)

    pad_i = jnp.arange(EP - E, dtype=jnp.int32)
    ci = et * NP + src
    ci = jnp.concatenate([ci, pad_i % NP])
    dst_p = jnp.concatenate([dst, N + pad_i % (NP - N)])

    x_p = jnp.pad(x, ((0, NP - N), (0, 0)))
    seg_p = jnp.concatenate(
        [segment_ids.astype(jnp.int32), jnp.full((NP - N,), B, jnp.int32)]
    ).reshape(1, NP)
    deg = _deg(dst_p)
    zeros_init = jnp.zeros((NP, D), jnp.float32)

    h = x_p
    hw = _transform(h, W_rel[0])
    acc = _edge_pass(hw.reshape(R * NP, D), ci.reshape(NW, C, K), dst_p,
                     zeros_init, deg)
    for l in range(1, L):
        hw, h = _combine_transform(acc.reshape(NC, NP, D), h, W_self[l - 1],
                                   deg, W_rel[l])
        acc = _edge_pass(hw.reshape(R * NP, D), ci.reshape(NW, C, K), dst_p,
                         zeros_init, deg)

    wcat = jnp.pad(jnp.stack([fc_w[:D, 0], fc_w[D:2 * D, 0],
                              fc_w[2 * D:3 * D, 0]], axis=1),
                   ((0, 0), (0, D - 3)))
    wr = fc_w[3 * D:, 0].reshape(1, RELD)
    out = _tail(acc.reshape(NC, NP, D), h, W_self[L - 1], deg, seg_p,
                head_ids.astype(jnp.int32).reshape(B, 1),
                tail_ids.astype(jnp.int32).reshape(B, 1),
                rel_labels.astype(jnp.int32).reshape(B, 1),
                rel_emb, wr, wcat, fc_b.reshape(1, 1))
    return out


# speedup vs baseline: 1.2593x; 1.0039x over previous
import functools

import jax
import jax.numpy as jnp
from jax import lax
from jax.experimental import pallas as pl
from jax.experimental.pallas import tpu as pltpu
from jax.experimental.pallas import tpu_sc as plsc

N = 10000
E = 320000
D = 128
R = 8
RELD = 32
B = 100
L = 2

NC = 2
NS = 16
NW = NC * NS

NP = 10240
K = 128
C = 80
EPW = C * K
EP = NW * EPW

_MESH = plsc.VectorSubcoreMesh(core_axis_name="c", subcore_axis_name="s")


@functools.partial(
    pl.kernel,
    out_type=jax.ShapeDtypeStruct((NW, NP), jnp.float32),
    mesh=_MESH,
    scratch_types=[
        pltpu.VMEM((EPW,), jnp.int32),
        pltpu.VMEM((NP,), jnp.float32),
    ],
    compiler_params=pltpu.CompilerParams(needs_layout_passes=False),
)
def _deg(dst_hbm, out_hbm, dst_v, deg_v):
    cid = lax.axis_index("c")
    sid = lax.axis_index("s")
    wid = sid * NC + cid
    pltpu.sync_copy(dst_hbm.at[pl.ds(wid * EPW, EPW)], dst_v)

    zeros = jnp.zeros((16,), jnp.float32)
    def zbody(i, _):
        deg_v[pl.ds(pl.multiple_of(i * 16, 16), 16)] = zeros
        return 0
    lax.fori_loop(0, NP // 16, zbody, 0)

    ones = jnp.ones((16,), jnp.float32)
    def body(i, _):
        idx = dst_v[pl.ds(pl.multiple_of(i * 16, 16), 16)]
        plsc.addupdate_scatter(deg_v, [idx], ones)
        return 0
    lax.fori_loop(0, EPW // 16, body, 0)

    pltpu.sync_copy(deg_v, out_hbm.at[wid])


@functools.partial(
    pl.kernel,
    out_type=jax.ShapeDtypeStruct((NC * NP, D), jnp.float32),
    mesh=_MESH,
    scratch_types=[
        pltpu.VMEM((C, K), jnp.int32),
        pltpu.VMEM((K,), jnp.int32),
        pltpu.VMEM((K,), jnp.int32),
        pltpu.VMEM((K, D), jnp.float32),
        pltpu.VMEM((K, D), jnp.float32),
        pltpu.VMEM_SHARED((NP, D), jnp.float32),
        pltpu.SemaphoreType.DMA,
        pltpu.SemaphoreType.DMA,
        pltpu.SemaphoreType.DMA,
        pltpu.SemaphoreType.DMA,
        pltpu.SemaphoreType.DMA,
    ],
)
def _edge_pass(table_hbm, ci_hbm, dst_hbm, zeros_hbm, deg_hbm, out_hbm,
               ci_v, dst0_v, dst1_v, rows0_v, rows1_v, acc_sh,
               semg0, semg1, semd0, semd1, semz):
    del deg_hbm
    cid = lax.axis_index("c")
    sid = lax.axis_index("s")
    wid = sid * NC + cid
    rpt = NP // NS

    pltpu.async_copy(zeros_hbm.at[pl.ds(sid * rpt, rpt)],
                     acc_sh.at[pl.ds(sid * rpt, rpt)], semz)
    pltpu.sync_copy(ci_hbm.at[wid], ci_v)

    rows = (rows0_v, rows1_v)
    dsts = (dst0_v, dst1_v)
    semg = (semg0, semg1)
    semd = (semd0, semd1)
    base = wid * EPW

    def start(j, b):
        pltpu.async_copy(dst_hbm.at[pl.ds(base + j * K, K)], dsts[b], semd[b])
        pltpu.async_copy(table_hbm.at[ci_v.at[j]], rows[b], semg[b])

    def finish(j, b):
        pltpu.make_async_copy(dst_hbm.at[pl.ds(base + j * K, K)], dsts[b],
                              semd[b]).wait()
        pltpu.make_async_copy(table_hbm.at[ci_v.at[j]], rows[b],
                              semg[b]).wait()
        pltpu.sync_copy(rows[b], acc_sh.at[dsts[b]], add=True)

    start(0, 0)
    pltpu.make_async_copy(zeros_hbm.at[pl.ds(sid * rpt, rpt)],
                          acc_sh.at[pl.ds(sid * rpt, rpt)], semz).wait()
    plsc.subcore_barrier()

    def body(i, _):
        jj = i * 2
        for b in range(2):
            j = jj + b

            @pl.when(j + 1 < C)
            def _():
                start(j + 1, 1 - b)

            finish(j, b)
        return 0
    lax.fori_loop(0, C // 2, body, 0)

    plsc.subcore_barrier()
    pltpu.sync_copy(acc_sh.at[pl.ds(sid * rpt, rpt)],
                    out_hbm.at[pl.ds(cid * NP + sid * rpt, rpt)])


BN = 5120
NB = NP // BN


def _transform_body(h_ref, w_ref, out_ref):
    out_ref[0] = jnp.dot(h_ref[...], w_ref[0],
                         preferred_element_type=jnp.float32)


TN = 10240
_transform = pl.pallas_call(
    _transform_body,
    grid=(NP // TN, R),
    in_specs=[
        pl.BlockSpec((TN, D), lambda i, r: (i, 0)),
        pl.BlockSpec((1, D, D), lambda i, r: (r, 0, 0)),
    ],
    out_specs=pl.BlockSpec((1, TN, D), lambda i, r: (r, i, 0)),
    out_shape=jax.ShapeDtypeStruct((R, NP, D), jnp.float32),
)


def _combine_transform_body(acc_ref, h_ref, wself_ref, deg_ref, wrel_ref,
                            hw_ref, h1_ref):
    r = pl.program_id(0)

    @pl.when(r == 0)
    def _():
        deg = jnp.sum(deg_ref[...], axis=0)
        norm = 1.0 / jnp.where(deg == 0.0, 1.0, deg)
        a = acc_ref[0] + acc_ref[1]
        hw = jnp.dot(h_ref[...], wself_ref[...],
                     preferred_element_type=jnp.float32)
        h1_ref[...] = jnp.maximum(a * norm[:, None] + hw, 0.0)

    hw_ref[0] = jnp.dot(h1_ref[...], wrel_ref[0],
                        preferred_element_type=jnp.float32)


_combine_transform = pl.pallas_call(
    _combine_transform_body,
    grid=(R,),
    in_specs=[
        pl.BlockSpec((NC, NP, D), lambda r: (0, 0, 0)),
        pl.BlockSpec((NP, D), lambda r: (0, 0)),
        pl.BlockSpec((D, D), lambda r: (0, 0)),
        pl.BlockSpec((NW, NP), lambda r: (0, 0)),
        pl.BlockSpec((1, D, D), lambda r: (r, 0, 0)),
    ],
    out_specs=[
        pl.BlockSpec((1, NP, D), lambda r: (r, 0, 0)),
        pl.BlockSpec((NP, D), lambda r: (0, 0)),
    ],
    out_shape=[
        jax.ShapeDtypeStruct((R, NP, D), jnp.float32),
        jax.ShapeDtypeStruct((NP, D), jnp.float32),
    ],
)


def _tail_body(acc_ref, h_ref, wself_ref, deg_ref, seg_ref, head_ref,
               tail_ref, rlab_ref, rel_emb_ref, wr_ref, wcat_ref, fcb_ref,
               out_ref, sg, sh, st, sc):
    i = pl.program_id(0)
    deg = jnp.sum(deg_ref[...], axis=0)
    norm = 1.0 / jnp.where(deg == 0.0, 1.0, deg)
    a = acc_ref[0] + acc_ref[1]
    hw = jnp.dot(h_ref[...], wself_ref[...],
                 preferred_element_type=jnp.float32)
    repr_blk = jnp.maximum(a * norm[:, None] + hw, 0.0)
    u = jnp.dot(repr_blk, wcat_ref[...],
                preferred_element_type=jnp.float32)
    n_iota = lax.broadcasted_iota(jnp.int32, (B, BN), 1) + i * BN
    b_iota = lax.broadcasted_iota(jnp.int32, (B, BN), 0)
    segm = (seg_ref[...] == b_iota).astype(jnp.float32)
    headm = (head_ref[...] == n_iota).astype(jnp.float32)
    tailm = (tail_ref[...] == n_iota).astype(jnp.float32)
    pg = jnp.dot(segm, u, preferred_element_type=jnp.float32)
    ph = jnp.dot(headm, u, preferred_element_type=jnp.float32)
    pt = jnp.dot(tailm, u, preferred_element_type=jnp.float32)
    cnt = jnp.sum(segm, axis=1, keepdims=True)

    @pl.when(i == 0)
    def _():
        sg[...] = pg
        sh[...] = ph
        st[...] = pt
        sc[...] = cnt

    @pl.when(i > 0)
    def _():
        sg[...] += pg
        sh[...] += ph
        st[...] += pt
        sc[...] += cnt

    @pl.when(i == NB - 1)
    def _():
        r_iota = lax.broadcasted_iota(jnp.int32, (B, R), 1)
        relm = (rlab_ref[...] == r_iota).astype(jnp.float32)
        z = jnp.sum(rel_emb_ref[...] * wr_ref[...], axis=1,
                    keepdims=True)
        rp = jnp.dot(relm, z, preferred_element_type=jnp.float32)
        counts = jnp.maximum(sc[...], 1.0)
        out_ref[...] = sg[:, 0:1] / counts + sh[:, 1:2] + st[:, 2:3] + rp \
            + fcb_ref[...]


_tail = pl.pallas_call(
    _tail_body,
    grid=(NB,),
    in_specs=[
        pl.BlockSpec((NC, BN, D), lambda i: (0, i, 0)),
        pl.BlockSpec((BN, D), lambda i: (i, 0)),
        pl.BlockSpec((D, D), lambda i: (0, 0)),
        pl.BlockSpec((NW, BN), lambda i: (0, i)),
        pl.BlockSpec((1, BN), lambda i: (0, i)),
        pl.BlockSpec((B, 1), lambda i: (0, 0)),
        pl.BlockSpec((B, 1), lambda i: (0, 0)),
        pl.BlockSpec((B, 1), lambda i: (0, 0)),
        pl.BlockSpec((R, RELD), lambda i: (0, 0)),
        pl.BlockSpec((1, RELD), lambda i: (0, 0)),
        pl.BlockSpec((D, 128), lambda i: (0, 0)),
        pl.BlockSpec((1, 1), lambda i: (0, 0)),
    ],
    out_specs=pl.BlockSpec((B, 1), lambda i: (0, 0)),
    out_shape=jax.ShapeDtypeStruct((B, 1), jnp.float32),
    scratch_shapes=[
        pltpu.VMEM((B, 128), jnp.float32),
        pltpu.VMEM((B, 128), jnp.float32),
        pltpu.VMEM((B, 128), jnp.float32),
        pltpu.VMEM((B, 1), jnp.float32),
    ],
)


def kernel(x, edge_index, edge_type, segment_ids, head_ids, tail_ids,
           rel_labels, W_rel, W_self, rel_emb, fc_w, fc_b):
    src = edge_index[0].astype(jnp.int32)
    dst = edge_index[1].astype(jnp.int32)
    et = edge_type.astype(jnp.int32)

    pad_i = jnp.arange(EP - E, dtype=jnp.int32)
    ci = et * NP + src
    ci = jnp.concatenate([ci, pad_i % NP])
    dst_p = jnp.concatenate([dst, N + pad_i % (NP - N)])

    x_p = jnp.pad(x, ((0, NP - N), (0, 0)))
    seg_p = jnp.concatenate(
        [segment_ids.astype(jnp.int32), jnp.full((NP - N,), B, jnp.int32)]
    ).reshape(1, NP)
    deg = _deg(dst_p)
    zeros_init = jnp.zeros((NP, D), jnp.float32)

    ci3 = ci.reshape(NW, C, K)

    h = x_p
    hw = _transform(h, W_rel[0])
    acc = _edge_pass(hw.reshape(R * NP, D), ci3, dst_p, zeros_init, deg)
    for l in range(1, L):
        hw, h = _combine_transform(acc.reshape(NC, NP, D), h, W_self[l - 1],
                                   deg, W_rel[l])
        acc = _edge_pass(hw.reshape(R * NP, D), ci3, dst_p, zeros_init, deg)

    wcat = jnp.pad(jnp.stack([fc_w[:D, 0], fc_w[D:2 * D, 0],
                              fc_w[2 * D:3 * D, 0]], axis=1),
                   ((0, 0), (0, D - 3)))
    wr = fc_w[3 * D:, 0].reshape(1, RELD)
    out = _tail(acc.reshape(NC, NP, D), h, W_self[L - 1], deg, seg_p,
                head_ids.astype(jnp.int32).reshape(B, 1),
                tail_ids.astype(jnp.int32).reshape(B, 1),
                rel_labels.astype(jnp.int32).reshape(B, 1),
                rel_emb, wr, wcat, fc_b.reshape(1, 1))
    return out
